# Initial kernel scaffold; baseline (speedup 1.0000x reference)
import jax
import jax.numpy as jnp
from jax.experimental import pallas as pl


def kernel(x, edge_index, Wl1, Wr1, att1, b1, Wl3, Wr3, att3, b3, Wl2, Wr2, att2, b2):
    raise NotImplementedError("write your pallas kernel here")


# speedup vs baseline: 10.0142x; 10.0142x over previous
import functools

import jax
import jax.numpy as jnp
from jax import lax
from jax.experimental import pallas as pl
from jax.experimental.pallas import tpu as pltpu
from jax.experimental.pallas import tpu_sc as plsc

N_NODES = 10000
N_EDGES = 320000 + N_NODES
NC = 2
NS = 16
B = 64
B3 = 128
EPT12 = 20736
E_PAD = NS * EPT12
NBLK12 = EPT12 // B
EPT3 = E_PAD // (NC * NS)
NBLK3 = EPT3 // B3
RPT = 640
N_PAD = NS * RPT
DROWS = N_PAD // 32
AROWS3 = N_PAD // 8
MROWS = 1000
GRID = N_NODES // MROWS

_params = pltpu.CompilerParams(needs_layout_passes=False)


def _mesh():
    return plsc.VectorSubcoreMesh(core_axis_name="c", subcore_axis_name="s",
                                  num_cores=NC, num_subcores=NS)


def _lrelu(v):
    return jnp.maximum(v, 0.0) + 0.2 * jnp.minimum(v, 0.0)


def _edges8_body(xl_hbm, xr_hbm, src_hbm, dst_hbm, att_hbm, acc_hbm,
                 src_v, dst_v, gl_idx, gr_idx, gd_idx, xl_rows, xr_rows,
                 stag, stag_den, ex_buf, att_flat, att_cols, denbuf, rden,
                 msg_sh, den_sh, sem1, sem2):
    cid = lax.axis_index("c")
    sid = lax.axis_index("s")
    iota = lax.iota(jnp.int32, 16)
    e4 = iota >> 2
    h4 = iota & 3
    z16 = jnp.zeros((16,), jnp.float32)
    r0 = sid * RPT

    pltpu.sync_copy(att_hbm, att_flat)
    for cc in range(32):
        g = plsc.load_gather(att_flat, [(cid * 4 + h4) * 32 + cc])
        plsc.store_scatter(att_cols, [cc * 16 + iota], g)

    def _zrow(r, _):
        for k in range(8):
            plsc.store_scatter(stag, [r + 0 * iota, k * 16 + iota], z16)
            plsc.store_scatter(stag_den, [r + 0 * iota, k * 16 + iota], z16)
        return 0
    lax.fori_loop(0, B, _zrow, 0)

    def _zacc(g, _):
        pltpu.sync_copy(stag, msg_sh.at[pl.ds(r0 + g * B, B)])
        return 0
    lax.fori_loop(0, RPT // B, _zacc, 0)
    pltpu.sync_copy(stag.at[pl.ds(0, DROWS // NS)],
                    den_sh.at[pl.ds(sid * (DROWS // NS), DROWS // NS)])
    plsc.subcore_barrier()

    def _block(g, _):
        base = sid * EPT12 + g * B
        pltpu.sync_copy(src_hbm.at[pl.ds(base, B)], src_v)
        pltpu.sync_copy(dst_hbm.at[pl.ds(base, B)], dst_v)
        for k in range(B // 16):
            sl = pl.ds(k * 16, 16)
            gl_idx[sl] = src_v[sl] * 2 + cid
            gr_idx[sl] = dst_v[sl] * 2 + cid
            gd_idx[sl] = dst_v[sl] >> 5
        cp1 = pltpu.async_copy(xl_hbm.at[gl_idx], xl_rows, sem1)
        cp2 = pltpu.async_copy(xr_hbm.at[gr_idx], xr_rows, sem2)
        cp1.wait()
        cp2.wait()

        def _quad(q, _):
            rowq = q * 4 + e4
            acc_a = z16
            for cc in range(32):
                colv = h4 * 32 + cc
                xlv = plsc.load_gather(xl_rows, [rowq, colv])
                xrv = plsc.load_gather(xr_rows, [rowq, colv])
                acc_a = acc_a + _lrelu(xlv + xrv) * att_cols[pl.ds(cc * 16, 16)]
            ids = base + q * 4 + e4
            exv = jnp.where(ids < N_EDGES, jnp.exp(acc_a), 0.0)
            plsc.store_scatter(ex_buf, [q * 16 + iota], exv)
            for e in range(4):
                row = q * 4 + e
                for hk in range(8):
                    w = plsc.load_gather(
                        ex_buf, [q * 16 + e * 4 + hk // 2 + 0 * iota])
                    xlc = plsc.load_gather(
                        xl_rows, [row + 0 * iota, hk * 16 + iota])
                    plsc.store_scatter(
                        stag, [row + 0 * iota, hk * 16 + iota], w * xlc)
            return 0
        lax.fori_loop(0, B // 4, _quad, 0)
        pltpu.sync_copy(stag, msg_sh.at[dst_v], add=True)

        for k in range(B // 16):
            rows = k * 16 + iota
            colv = (dst_v[pl.ds(k * 16, 16)] & 31) * 4
            for h in range(4):
                exh = plsc.load_gather(ex_buf, [rows * 4 + h])
                plsc.store_scatter(stag_den, [rows, colv + h], exh)
        pltpu.sync_copy(stag_den, den_sh.at[gd_idx], add=True)
        for k in range(B // 16):
            rows = k * 16 + iota
            colv = (dst_v[pl.ds(k * 16, 16)] & 31) * 4
            for h in range(4):
                plsc.store_scatter(stag_den, [rows, colv + h], z16)
        return 0
    lax.fori_loop(0, NBLK12, _block, 0)
    plsc.subcore_barrier()

    pltpu.sync_copy(den_sh.at[pl.ds(sid * (DROWS // NS), DROWS // NS)],
                    denbuf)

    def _dred(q, _):
        idx = q * 16 + iota
        v = plsc.load_gather(denbuf, [idx >> 7, idx & 127])
        rden[pl.ds(q * 16, 16)] = 1.0 / (v + 1e-16)
        return 0
    lax.fori_loop(0, 4 * RPT // 16, _dred, 0)

    def _norm(g, _):
        pltpu.sync_copy(msg_sh.at[pl.ds(r0 + g * B, B)], stag)
        def _nrow(nn, _):
            for hk in range(8):
                w = plsc.load_gather(rden, [(g * B + nn) * 4 + hk // 2
                                            + 0 * iota])
                m = plsc.load_gather(stag, [nn + 0 * iota, hk * 16 + iota])
                plsc.store_scatter(stag, [nn + 0 * iota, hk * 16 + iota],
                                   m * w)
            return 0
        lax.fori_loop(0, B, _nrow, 0)
        pltpu.sync_copy(stag, acc_hbm.at[cid, pl.ds(r0 + g * B, B)])
        return 0
    lax.fori_loop(0, RPT // B, _norm, 0)


_edges8 = functools.partial(
    pl.kernel,
    out_type=jax.ShapeDtypeStruct((NC, N_PAD, 128), jnp.float32),
    scratch_types=[
        pltpu.VMEM((B,), jnp.int32),
        pltpu.VMEM((B,), jnp.int32),
        pltpu.VMEM((B,), jnp.int32),
        pltpu.VMEM((B,), jnp.int32),
        pltpu.VMEM((B,), jnp.int32),
        pltpu.VMEM((B, 128), jnp.float32),
        pltpu.VMEM((B, 128), jnp.float32),
        pltpu.VMEM((B, 128), jnp.float32),
        pltpu.VMEM((B, 128), jnp.float32),
        pltpu.VMEM((4 * B,), jnp.float32),
        pltpu.VMEM((256,), jnp.float32),
        pltpu.VMEM((512,), jnp.float32),
        pltpu.VMEM((DROWS // NS, 128), jnp.float32),
        pltpu.VMEM((4 * RPT,), jnp.float32),
        pltpu.VMEM_SHARED((N_PAD, 128), jnp.float32),
        pltpu.VMEM_SHARED((DROWS, 128), jnp.float32),
        pltpu.SemaphoreType.DMA,
        pltpu.SemaphoreType.DMA,
    ],
    compiler_params=_params,
)


def _edges1_body(xl_hbm, xr_hbm, src_hbm, dst_hbm, att_hbm, acc_hbm,
                 xl3_v, xr3_v, src_v, dst_v, ga_idx, stag3, att2_v, acc_sh,
                 sem1):
    cid = lax.axis_index("c")
    sid = lax.axis_index("s")
    wid = sid * NC + cid
    iota = lax.iota(jnp.int32, 16)
    z16 = jnp.zeros((16,), jnp.float32)
    apt = AROWS3 // NS

    pltpu.sync_copy(xl_hbm, xl3_v)
    pltpu.sync_copy(xr_hbm, xr3_v)
    pltpu.sync_copy(att_hbm, att2_v)
    a20 = plsc.load_gather(att2_v, [0 * iota])
    a21 = plsc.load_gather(att2_v, [0 * iota + 1])

    def _zrow(r, _):
        for k in range(8):
            plsc.store_scatter(stag3, [r + 0 * iota, k * 16 + iota], z16)
        return 0
    lax.fori_loop(0, B3, _zrow, 0)

    pltpu.sync_copy(stag3.at[pl.ds(0, apt)],
                    acc_sh.at[pl.ds(sid * apt, apt)])
    plsc.subcore_barrier()

    def _block(g, _):
        base = wid * EPT3 + g * B3
        pltpu.sync_copy(src_hbm.at[pl.ds(base, B3)], src_v)
        pltpu.sync_copy(dst_hbm.at[pl.ds(base, B3)], dst_v)
        for k in range(B3 // 16):
            sl = pl.ds(k * 16, 16)
            ga_idx[sl] = dst_v[sl] >> 3
        for k in range(B3 // 16):
            sl = pl.ds(k * 16, 16)
            sv = src_v[sl]
            dv = dst_v[sl]
            xl0 = plsc.load_gather(xl3_v, [sv * 2])
            xl1 = plsc.load_gather(xl3_v, [sv * 2 + 1])
            xr0 = plsc.load_gather(xr3_v, [dv * 2])
            xr1 = plsc.load_gather(xr3_v, [dv * 2 + 1])
            alpha = a20 * _lrelu(xl0 + xr0) + a21 * _lrelu(xl1 + xr1)
            ids = base + k * 16 + iota
            ex = jnp.where(ids < N_EDGES, jnp.exp(alpha), 0.0)
            rows = k * 16 + iota
            colv = (dv & 7) * 16
            plsc.store_scatter(stag3, [rows, colv], ex * xl0)
            plsc.store_scatter(stag3, [rows, colv + 1], ex * xl1)
            plsc.store_scatter(stag3, [rows, colv + 2], ex)
        pltpu.sync_copy(stag3, acc_sh.at[ga_idx], add=True)
        for k in range(B3 // 16):
            sl = pl.ds(k * 16, 16)
            rows = k * 16 + iota
            colv = (dst_v[sl] & 7) * 16
            plsc.store_scatter(stag3, [rows, colv], z16)
            plsc.store_scatter(stag3, [rows, colv + 1], z16)
            plsc.store_scatter(stag3, [rows, colv + 2], z16)
        return 0
    lax.fori_loop(0, NBLK3, _block, 0)
    plsc.subcore_barrier()

    pltpu.sync_copy(acc_sh.at[pl.ds(sid * apt, apt)],
                    acc_hbm.at[cid, pl.ds(sid * apt, apt)])


_edges1 = functools.partial(
    pl.kernel,
    out_type=jax.ShapeDtypeStruct((NC, AROWS3, 128), jnp.float32),
    scratch_types=[
        pltpu.VMEM((2 * N_NODES,), jnp.float32),
        pltpu.VMEM((2 * N_NODES,), jnp.float32),
        pltpu.VMEM((B3,), jnp.int32),
        pltpu.VMEM((B3,), jnp.int32),
        pltpu.VMEM((B3,), jnp.int32),
        pltpu.VMEM((B3, 128), jnp.float32),
        pltpu.VMEM((16,), jnp.float32),
        pltpu.VMEM_SHARED((AROWS3, 128), jnp.float32),
        pltpu.SemaphoreType.DMA,
    ],
    compiler_params=_params,
)


def _mm2_body(x_ref, wl_ref, wr_ref, xl_ref, xr_ref):
    xb = x_ref[...]
    xl_ref[...] = jnp.dot(xb, wl_ref[...], preferred_element_type=jnp.float32)
    xr_ref[...] = jnp.dot(xb, wr_ref[...], preferred_element_type=jnp.float32)


def _mid_body(a0_ref, a1_ref, b_ref, wl_ref, wr_ref, xl_ref, xr_ref):
    h = jnp.concatenate([a0_ref[...], a1_ref[...]], axis=1) + b_ref[...]
    h = jnp.where(h > 0, h, jnp.exp(h) - 1.0)
    xl_ref[...] = jnp.dot(h, wl_ref[...], preferred_element_type=jnp.float32)
    xr_ref[...] = jnp.dot(h, wr_ref[...], preferred_element_type=jnp.float32)


def _fin_body(a0_ref, a1_ref, b_ref, o_ref):
    s = a0_ref[...] + a1_ref[...]
    z = s[:, 0:2] / (s[:, 2:3] + 1e-16) + b_ref[:, 0:2]
    m = jnp.max(z, axis=1, keepdims=True)
    ls = m + jnp.log(jnp.sum(jnp.exp(z - m), axis=1, keepdims=True))
    o = z - ls
    o_ref[...] = jnp.concatenate(
        [o, jnp.zeros((o.shape[0], 14), jnp.float32)], axis=1)


def _row_spec(w):
    return pl.BlockSpec((MROWS, w), lambda i: (i, 0))


def _full_spec(r, c):
    return pl.BlockSpec((r, c), lambda i: (0, 0))


def _mm2(x, wl, wr, kdim):
    f = pl.pallas_call(
        _mm2_body,
        grid=(GRID,),
        in_specs=[_row_spec(kdim), _full_spec(kdim, 256), _full_spec(kdim, 256)],
        out_specs=[_row_spec(256), _row_spec(256)],
        out_shape=[jax.ShapeDtypeStruct((N_NODES, 256), jnp.float32)] * 2,
    )
    return f(x, wl, wr)


def _mid(a0, a1, b, wl, wr):
    f = pl.pallas_call(
        _mid_body,
        grid=(GRID,),
        in_specs=[_row_spec(128), _row_spec(128), _full_spec(1, 256),
                  _full_spec(256, 256), _full_spec(256, 256)],
        out_specs=[_row_spec(256), _row_spec(256)],
        out_shape=[jax.ShapeDtypeStruct((N_NODES, 256), jnp.float32)] * 2,
    )
    return f(a0, a1, b, wl, wr)


def _fin(a0, a1, b):
    f = pl.pallas_call(
        _fin_body,
        grid=(GRID,),
        in_specs=[_row_spec(16), _row_spec(16), _full_spec(1, 16)],
        out_specs=_row_spec(16),
        out_shape=jax.ShapeDtypeStruct((N_NODES, 16), jnp.float32),
    )
    return f(a0, a1, b)


def kernel(x, edge_index, Wl1, Wr1, att1, b1, Wl3, Wr3, att3, b3,
           Wl2, Wr2, att2, b2):
    loop = jnp.arange(N_NODES, dtype=jnp.int32)
    pad = jnp.zeros((E_PAD - N_EDGES,), jnp.int32)
    src = jnp.concatenate([edge_index[0].astype(jnp.int32), loop, pad])
    dst = jnp.concatenate([edge_index[1].astype(jnp.int32), loop, pad])

    ek8 = _edges8(_edges8_body, mesh=_mesh())
    ek1 = _edges1(_edges1_body, mesh=_mesh())

    xl, xr = _mm2(x, Wl1, Wr1, 128)
    acc = ek8(xl.reshape(2 * N_NODES, 128), xr.reshape(2 * N_NODES, 128),
              src, dst, att1.reshape(-1))

    xl, xr = _mid(acc[0, :N_NODES], acc[1, :N_NODES],
                  b1.reshape(1, 256), Wl3, Wr3)
    acc = ek8(xl.reshape(2 * N_NODES, 128), xr.reshape(2 * N_NODES, 128),
              src, dst, att3.reshape(-1))

    wcat = jnp.concatenate(
        [jnp.pad(Wl2, ((0, 0), (0, 126))), jnp.pad(Wr2, ((0, 0), (0, 126)))],
        axis=1)
    xlr, _ = _mid(acc[0, :N_NODES], acc[1, :N_NODES],
                  b3.reshape(1, 256), wcat, wcat)
    xl3 = xlr[:, 0:2].reshape(-1)
    xr3 = xlr[:, 128:130].reshape(-1)
    acc3 = ek1(xl3, xr3, src, dst, jnp.pad(att2.reshape(-1), (0, 14)))
    acc3 = acc3.reshape(NC, N_PAD, 16)

    out = _fin(acc3[0, :N_NODES], acc3[1, :N_NODES],
               jnp.pad(b2, (0, 14)).reshape(1, 16))
    return out[:, :2]


# speedup vs baseline: 23.3543x; 2.3321x over previous
import functools

import jax
import jax.numpy as jnp
from jax import lax
from jax.experimental import pallas as pl
from jax.experimental.pallas import tpu as pltpu
from jax.experimental.pallas import tpu_sc as plsc

N_NODES = 10000
N_EDGES = 320000 + N_NODES
NC = 2
NS = 16
B = 64
B3 = 128
EPT12 = 20736
E_PAD = NS * EPT12
NBLK12 = EPT12 // B
EPT3 = E_PAD // (NC * NS)
NBLK3 = EPT3 // B3
RPT = 640
N_PAD = NS * RPT
DROWS = N_PAD // 32
AROWS3 = N_PAD // 8
MROWS = 1000
GRID = N_NODES // MROWS

_params = pltpu.CompilerParams(needs_layout_passes=False)


def _mesh():
    return plsc.VectorSubcoreMesh(core_axis_name="c", subcore_axis_name="s",
                                  num_cores=NC, num_subcores=NS)


def _lrelu(v):
    return jnp.maximum(v, 0.0) + 0.2 * jnp.minimum(v, 0.0)


def _edges8_body(xl_hbm, xr_hbm, src_hbm, dst_hbm, att_hbm, acc_hbm,
                 src_v, dst_v, gl_idx, gr_idx, gd_idx, xl_rows, xr_rows,
                 stag, stag_den, ex_buf, att_flat, denbuf, rden,
                 msg_sh, den_sh, sem1, sem2):
    cid = lax.axis_index("c")
    sid = lax.axis_index("s")
    iota = lax.iota(jnp.int32, 16)

    h4 = iota & 3
    z16 = jnp.zeros((16,), jnp.float32)
    r0 = sid * RPT

    pltpu.sync_copy(att_hbm, att_flat)
    attk = [att_flat[pl.ds(cid * 128 + k * 16, 16)] for k in range(8)]

    def _zrow(r, _):
        for k in range(8):
            plsc.store_scatter(stag, [r + 0 * iota, k * 16 + iota], z16)
            plsc.store_scatter(stag_den, [r + 0 * iota, k * 16 + iota], z16)
        return 0
    lax.fori_loop(0, B, _zrow, 0)

    def _zacc(g, _):
        pltpu.sync_copy(stag, msg_sh.at[pl.ds(r0 + g * B, B)])
        return 0
    lax.fori_loop(0, RPT // B, _zacc, 0)
    pltpu.sync_copy(stag.at[pl.ds(0, DROWS // NS)],
                    den_sh.at[pl.ds(sid * (DROWS // NS), DROWS // NS)])
    plsc.subcore_barrier()

    def _block(g, _):
        base = sid * EPT12 + g * B
        pltpu.sync_copy(src_hbm.at[pl.ds(base, B)], src_v)
        pltpu.sync_copy(dst_hbm.at[pl.ds(base, B)], dst_v)
        for k in range(B // 16):
            sl = pl.ds(k * 16, 16)
            gl_idx[sl] = src_v[sl] * 2 + cid
            gr_idx[sl] = dst_v[sl] * 2 + cid
            gd_idx[sl] = dst_v[sl] >> 5
        cp1 = pltpu.async_copy(xl_hbm.at[gl_idx], xl_rows, sem1)
        cp2 = pltpu.async_copy(xr_hbm.at[gr_idx], xr_rows, sem2)
        cp1.wait()
        cp2.wait()

        def _quad(q, _):
            for e in range(4):
                row = q * 4 + e
                ok = (base + row) < N_EDGES
                xls = []
                us = []
                for k in range(8):
                    xlk = xl_rows[row, pl.ds(k * 16, 16)]
                    xrk = xr_rows[row, pl.ds(k * 16, 16)]
                    xls.append(xlk)
                    t = _lrelu(xlk + xrk) * attk[k]
                    if k % 2 == 0:
                        us.append(t)
                    else:
                        us[k // 2] = us[k // 2] + t
                ws = []
                for h in range(4):
                    c = plsc.cumsum(us[h])
                    a_spl = jnp.take(c, jnp.full((16,), 15, jnp.int32))
                    ws.append(jnp.where(ok, jnp.exp(a_spl), 0.0))
                merged = jnp.where(h4 == 0, ws[0],
                                   jnp.where(h4 == 1, ws[1],
                                             jnp.where(h4 == 2, ws[2], ws[3])))
                plsc.store_scatter(ex_buf, [q * 16 + e * 4 + (iota & 3)],
                                   merged, mask=iota < 4)
                for k in range(8):
                    stag[row, pl.ds(k * 16, 16)] = ws[k // 2] * xls[k]
            return 0
        lax.fori_loop(0, B // 4, _quad, 0)
        pltpu.sync_copy(stag, msg_sh.at[dst_v], add=True)

        if True:
            for k in range(B // 16):
                rows = k * 16 + iota
                colv = (dst_v[pl.ds(k * 16, 16)] & 31) * 4
                for h in range(4):
                    exh = plsc.load_gather(ex_buf, [rows * 4 + h])
                    plsc.store_scatter(stag_den, [rows, colv + h], exh)
            pltpu.sync_copy(stag_den, den_sh.at[gd_idx], add=True)
            for k in range(B // 16):
                rows = k * 16 + iota
                colv = (dst_v[pl.ds(k * 16, 16)] & 31) * 4
                for h in range(4):
                    plsc.store_scatter(stag_den, [rows, colv + h], z16)
        return 0
    lax.fori_loop(0, NBLK12, _block, 0)
    plsc.subcore_barrier()

    pltpu.sync_copy(den_sh.at[pl.ds(sid * (DROWS // NS), DROWS // NS)],
                    denbuf)

    def _dred(q, _):
        idx = q * 16 + iota
        v = plsc.load_gather(denbuf, [idx >> 7, idx & 127])
        rden[pl.ds(q * 16, 16)] = 1.0 / (v + 1e-16)
        return 0
    lax.fori_loop(0, 4 * RPT // 16, _dred, 0)

    def _norm(g, _):
        pltpu.sync_copy(msg_sh.at[pl.ds(r0 + g * B, B)], stag)
        def _nrow(nn, _):
            for hk in range(8):
                w = plsc.load_gather(rden, [(g * B + nn) * 4 + hk // 2
                                            + 0 * iota])
                m = plsc.load_gather(stag, [nn + 0 * iota, hk * 16 + iota])
                plsc.store_scatter(stag, [nn + 0 * iota, hk * 16 + iota],
                                   m * w)
            return 0
        lax.fori_loop(0, B, _nrow, 0)
        pltpu.sync_copy(stag, acc_hbm.at[cid, pl.ds(r0 + g * B, B)])
        return 0
    lax.fori_loop(0, RPT // B, _norm, 0)


_edges8 = functools.partial(
    pl.kernel,
    out_type=jax.ShapeDtypeStruct((NC, N_PAD, 128), jnp.float32),
    scratch_types=[
        pltpu.VMEM((B,), jnp.int32),
        pltpu.VMEM((B,), jnp.int32),
        pltpu.VMEM((B,), jnp.int32),
        pltpu.VMEM((B,), jnp.int32),
        pltpu.VMEM((B,), jnp.int32),
        pltpu.VMEM((B, 128), jnp.float32),
        pltpu.VMEM((B, 128), jnp.float32),
        pltpu.VMEM((B, 128), jnp.float32),
        pltpu.VMEM((B, 128), jnp.float32),
        pltpu.VMEM((4 * B,), jnp.float32),
        pltpu.VMEM((256,), jnp.float32),
        pltpu.VMEM((DROWS // NS, 128), jnp.float32),
        pltpu.VMEM((4 * RPT,), jnp.float32),
        pltpu.VMEM_SHARED((N_PAD, 128), jnp.float32),
        pltpu.VMEM_SHARED((DROWS, 128), jnp.float32),
        pltpu.SemaphoreType.DMA,
        pltpu.SemaphoreType.DMA,
    ],
    compiler_params=_params,
)


def _edges1_body(xl_hbm, xr_hbm, src_hbm, dst_hbm, att_hbm, acc_hbm,
                 xl3_v, xr3_v, src_v, dst_v, ga_idx, stag3, att2_v, acc_sh,
                 sem1):
    cid = lax.axis_index("c")
    sid = lax.axis_index("s")
    wid = sid * NC + cid
    iota = lax.iota(jnp.int32, 16)
    z16 = jnp.zeros((16,), jnp.float32)
    apt = AROWS3 // NS

    pltpu.sync_copy(xl_hbm, xl3_v)
    pltpu.sync_copy(xr_hbm, xr3_v)
    pltpu.sync_copy(att_hbm, att2_v)
    a20 = plsc.load_gather(att2_v, [0 * iota])
    a21 = plsc.load_gather(att2_v, [0 * iota + 1])

    def _zrow(r, _):
        for k in range(8):
            plsc.store_scatter(stag3, [r + 0 * iota, k * 16 + iota], z16)
        return 0
    lax.fori_loop(0, B3, _zrow, 0)

    pltpu.sync_copy(stag3.at[pl.ds(0, apt)],
                    acc_sh.at[pl.ds(sid * apt, apt)])
    plsc.subcore_barrier()

    def _block(g, _):
        base = wid * EPT3 + g * B3
        pltpu.sync_copy(src_hbm.at[pl.ds(base, B3)], src_v)
        pltpu.sync_copy(dst_hbm.at[pl.ds(base, B3)], dst_v)
        for k in range(B3 // 16):
            sl = pl.ds(k * 16, 16)
            ga_idx[sl] = dst_v[sl] >> 3
        for k in range(B3 // 16):
            sl = pl.ds(k * 16, 16)
            sv = src_v[sl]
            dv = dst_v[sl]
            xl0 = plsc.load_gather(xl3_v, [sv * 2])
            xl1 = plsc.load_gather(xl3_v, [sv * 2 + 1])
            xr0 = plsc.load_gather(xr3_v, [dv * 2])
            xr1 = plsc.load_gather(xr3_v, [dv * 2 + 1])
            alpha = a20 * _lrelu(xl0 + xr0) + a21 * _lrelu(xl1 + xr1)
            ids = base + k * 16 + iota
            ex = jnp.where(ids < N_EDGES, jnp.exp(alpha), 0.0)
            rows = k * 16 + iota
            colv = (dv & 7) * 16
            plsc.store_scatter(stag3, [rows, colv], ex * xl0)
            plsc.store_scatter(stag3, [rows, colv + 1], ex * xl1)
            plsc.store_scatter(stag3, [rows, colv + 2], ex)
        pltpu.sync_copy(stag3, acc_sh.at[ga_idx], add=True)
        for k in range(B3 // 16):
            sl = pl.ds(k * 16, 16)
            rows = k * 16 + iota
            colv = (dst_v[sl] & 7) * 16
            plsc.store_scatter(stag3, [rows, colv], z16)
            plsc.store_scatter(stag3, [rows, colv + 1], z16)
            plsc.store_scatter(stag3, [rows, colv + 2], z16)
        return 0
    lax.fori_loop(0, NBLK3, _block, 0)
    plsc.subcore_barrier()

    pltpu.sync_copy(acc_sh.at[pl.ds(sid * apt, apt)],
                    acc_hbm.at[cid, pl.ds(sid * apt, apt)])


_edges1 = functools.partial(
    pl.kernel,
    out_type=jax.ShapeDtypeStruct((NC, AROWS3, 128), jnp.float32),
    scratch_types=[
        pltpu.VMEM((2 * N_NODES,), jnp.float32),
        pltpu.VMEM((2 * N_NODES,), jnp.float32),
        pltpu.VMEM((B3,), jnp.int32),
        pltpu.VMEM((B3,), jnp.int32),
        pltpu.VMEM((B3,), jnp.int32),
        pltpu.VMEM((B3, 128), jnp.float32),
        pltpu.VMEM((16,), jnp.float32),
        pltpu.VMEM_SHARED((AROWS3, 128), jnp.float32),
        pltpu.SemaphoreType.DMA,
    ],
    compiler_params=_params,
)


def _mm2_body(x_ref, wl_ref, wr_ref, xl_ref, xr_ref):
    xb = x_ref[...]
    xl_ref[...] = jnp.dot(xb, wl_ref[...], preferred_element_type=jnp.float32)
    xr_ref[...] = jnp.dot(xb, wr_ref[...], preferred_element_type=jnp.float32)


def _mid_body(a0_ref, a1_ref, b_ref, wl_ref, wr_ref, xl_ref, xr_ref):
    h = jnp.concatenate([a0_ref[...], a1_ref[...]], axis=1) + b_ref[...]
    h = jnp.where(h > 0, h, jnp.exp(h) - 1.0)
    xl_ref[...] = jnp.dot(h, wl_ref[...], preferred_element_type=jnp.float32)
    xr_ref[...] = jnp.dot(h, wr_ref[...], preferred_element_type=jnp.float32)


def _fin_body(a0_ref, a1_ref, b_ref, o_ref):
    s = a0_ref[...] + a1_ref[...]
    z = s[:, 0:2] / (s[:, 2:3] + 1e-16) + b_ref[:, 0:2]
    m = jnp.max(z, axis=1, keepdims=True)
    ls = m + jnp.log(jnp.sum(jnp.exp(z - m), axis=1, keepdims=True))
    o = z - ls
    o_ref[...] = jnp.concatenate(
        [o, jnp.zeros((o.shape[0], 14), jnp.float32)], axis=1)


def _row_spec(w):
    return pl.BlockSpec((MROWS, w), lambda i: (i, 0))


def _full_spec(r, c):
    return pl.BlockSpec((r, c), lambda i: (0, 0))


def _mm2(x, wl, wr, kdim):
    f = pl.pallas_call(
        _mm2_body,
        grid=(GRID,),
        in_specs=[_row_spec(kdim), _full_spec(kdim, 256), _full_spec(kdim, 256)],
        out_specs=[_row_spec(256), _row_spec(256)],
        out_shape=[jax.ShapeDtypeStruct((N_NODES, 256), jnp.float32)] * 2,
    )
    return f(x, wl, wr)


def _mid(a0, a1, b, wl, wr):
    f = pl.pallas_call(
        _mid_body,
        grid=(GRID,),
        in_specs=[_row_spec(128), _row_spec(128), _full_spec(1, 256),
                  _full_spec(256, 256), _full_spec(256, 256)],
        out_specs=[_row_spec(256), _row_spec(256)],
        out_shape=[jax.ShapeDtypeStruct((N_NODES, 256), jnp.float32)] * 2,
    )
    return f(a0, a1, b, wl, wr)


def _fin(a0, a1, b):
    f = pl.pallas_call(
        _fin_body,
        grid=(GRID,),
        in_specs=[_row_spec(16), _row_spec(16), _full_spec(1, 16)],
        out_specs=_row_spec(16),
        out_shape=jax.ShapeDtypeStruct((N_NODES, 16), jnp.float32),
    )
    return f(a0, a1, b)


def kernel(x, edge_index, Wl1, Wr1, att1, b1, Wl3, Wr3, att3, b3,
           Wl2, Wr2, att2, b2):
    loop = jnp.arange(N_NODES, dtype=jnp.int32)
    pad = jnp.zeros((E_PAD - N_EDGES,), jnp.int32)
    src = jnp.concatenate([edge_index[0].astype(jnp.int32), loop, pad])
    dst = jnp.concatenate([edge_index[1].astype(jnp.int32), loop, pad])

    ek8 = _edges8(_edges8_body, mesh=_mesh())
    ek1 = _edges1(_edges1_body, mesh=_mesh())

    xl, xr = _mm2(x, Wl1, Wr1, 128)
    acc = ek8(xl.reshape(2 * N_NODES, 128), xr.reshape(2 * N_NODES, 128),
              src, dst, att1.reshape(-1))

    xl, xr = _mid(acc[0, :N_NODES], acc[1, :N_NODES],
                  b1.reshape(1, 256), Wl3, Wr3)
    acc = ek8(xl.reshape(2 * N_NODES, 128), xr.reshape(2 * N_NODES, 128),
              src, dst, att3.reshape(-1))

    wcat = jnp.concatenate(
        [jnp.pad(Wl2, ((0, 0), (0, 126))), jnp.pad(Wr2, ((0, 0), (0, 126)))],
        axis=1)
    xlr, _ = _mid(acc[0, :N_NODES], acc[1, :N_NODES],
                  b3.reshape(1, 256), wcat, wcat)
    xl3 = xlr[:, 0:2].reshape(-1)
    xr3 = xlr[:, 128:130].reshape(-1)
    acc3 = ek1(xl3, xr3, src, dst, jnp.pad(att2.reshape(-1), (0, 14)))
    acc3 = acc3.reshape(NC, N_PAD, 16)

    out = _fin(acc3[0, :N_NODES], acc3[1, :N_NODES],
               jnp.pad(b2, (0, 14)).reshape(1, 16))
    return out[:, :2]


# speedup vs baseline: 33.7631x; 1.4457x over previous
import functools

import jax
import jax.numpy as jnp
from jax import lax
from jax.experimental import pallas as pl
from jax.experimental.pallas import tpu as pltpu
from jax.experimental.pallas import tpu_sc as plsc

N_NODES = 10000
N_EDGES = 320000 + N_NODES
NC = 2
NS = 16
B = 48
B3 = 128
CHB = 16
CHE = CHB * B
EPT12 = 20736
E_PAD = NS * EPT12
NBLK12 = EPT12 // B
EPT3 = E_PAD // (NC * NS)
NBLK3 = EPT3 // B3
RPT = 640
N_PAD = NS * RPT
DROWS = N_PAD // 32
AROWS3 = N_PAD // 8
MROWS = 1000
GRID = N_NODES // MROWS

_params = pltpu.CompilerParams(needs_layout_passes=False)


def _mesh():
    return plsc.VectorSubcoreMesh(core_axis_name="c", subcore_axis_name="s",
                                  num_cores=NC, num_subcores=NS)


def _lrelu(v):
    return jnp.maximum(v, 0.0) + 0.2 * jnp.minimum(v, 0.0)


def _edges8_body(xl_hbm, xr_hbm, src_hbm, dst_hbm, att_hbm, acc_hbm,
                 src_ch, dst_ch, gl_big, gr_big, gd_big, dst_sc,
                 xl_rows, xr_rows, stag, stag_den, ex_buf, att_flat,
                 msg_sh, den_sh, s_xl0, s_xr0, s_xl1, s_xr1):
    cid = lax.axis_index("c")
    sid = lax.axis_index("s")
    iota = lax.iota(jnp.int32, 16)
    h4 = iota & 3
    z16 = jnp.zeros((16,), jnp.float32)
    r0 = sid * RPT
    dpt = DROWS // NS

    pltpu.sync_copy(att_hbm, att_flat)
    attk = [att_flat[pl.ds(cid * 128 + k * 16, 16)] for k in range(8)]

    def _zrow(r, _):
        for k in range(8):
            plsc.store_scatter(stag, [r + 0 * iota, k * 16 + iota], z16)
            plsc.store_scatter(stag_den, [r + 0 * iota, k * 16 + iota], z16)
        return 0
    lax.fori_loop(0, B, _zrow, 0)

    def _zacc(g, _):
        pltpu.sync_copy(stag.at[pl.ds(0, 16)],
                        msg_sh.at[pl.ds(r0 + g * 16, 16)])
        return 0
    lax.fori_loop(0, RPT // 16, _zacc, 0)
    pltpu.sync_copy(stag.at[pl.ds(0, dpt)],
                    den_sh.at[pl.ds(sid * dpt, dpt)])
    plsc.subcore_barrier()

    def _refill(chunk):
        cb = sid * EPT12 + chunk * CHE
        pltpu.sync_copy(src_hbm.at[pl.ds(cb, CHE)], src_ch)
        pltpu.sync_copy(dst_hbm.at[pl.ds(cb, CHE)], dst_ch)

    def _prep(gn, pn):
        off = (gn % CHB) * B if isinstance(gn, int) else (gn & (CHB - 1)) * B
        for k in range(B // 16):
            sl = pl.ds(off + k * 16, 16)
            dl = pl.ds(k * 16, 16)
            sv = src_ch[sl]
            dv = dst_ch[sl]
            gl_big[pn, dl] = sv * 2 + cid
            gr_big[pn, dl] = dv * 2 + cid
            gd_big[pn, dl] = dv >> 5
            dst_sc[pn, dl] = dv

        @pl.when(pn == 0)
        def _():
            pltpu.async_copy(xl_hbm.at[gl_big.at[0]],
                             xl_rows.at[pl.ds(0, B)], s_xl0)
            pltpu.async_copy(xr_hbm.at[gr_big.at[0]],
                             xr_rows.at[pl.ds(0, B)], s_xr0)

        @pl.when(pn == 1)
        def _():
            pltpu.async_copy(xl_hbm.at[gl_big.at[1]],
                             xl_rows.at[pl.ds(B, B)], s_xl1)
            pltpu.async_copy(xr_hbm.at[gr_big.at[1]],
                             xr_rows.at[pl.ds(B, B)], s_xr1)

    def _wait(p):
        @pl.when(p == 0)
        def _():
            pltpu.make_async_copy(xl_hbm.at[gl_big.at[0]],
                                  xl_rows.at[pl.ds(0, B)], s_xl0).wait()
            pltpu.make_async_copy(xr_hbm.at[gr_big.at[0]],
                                  xr_rows.at[pl.ds(0, B)], s_xr0).wait()

        @pl.when(p == 1)
        def _():
            pltpu.make_async_copy(xl_hbm.at[gl_big.at[1]],
                                  xl_rows.at[pl.ds(B, B)], s_xl1).wait()
            pltpu.make_async_copy(xr_hbm.at[gr_big.at[1]],
                                  xr_rows.at[pl.ds(B, B)], s_xr1).wait()

    _refill(0)
    _prep(0, 0)

    def _block(g, _):
        p = g & 1
        gn = g + 1

        @pl.when(gn < NBLK12)
        def _():
            @pl.when((gn & (CHB - 1)) == 0)
            def _():
                _refill(gn // CHB)
            _prep(gn, gn & 1)

        _wait(p)
        base = sid * EPT12 + g * B

        def _quad(q, _):
            for e in range(4):
                row = q * 4 + e
                lrow = p * B + row
                ok = (base + row) < N_EDGES
                xls = []
                us = []
                for k in range(8):
                    xlk = xl_rows[lrow, pl.ds(k * 16, 16)]
                    xrk = xr_rows[lrow, pl.ds(k * 16, 16)]
                    xls.append(xlk)
                    t = _lrelu(xlk + xrk) * attk[k]
                    if k % 2 == 0:
                        us.append(t)
                    else:
                        us[k // 2] = us[k // 2] + t
                ws = []
                for h in range(4):
                    c = plsc.cumsum(us[h])
                    a_spl = jnp.take(c, jnp.full((16,), 15, jnp.int32))
                    ws.append(jnp.where(ok, jnp.exp(a_spl), 0.0))
                merged = jnp.where(h4 == 0, ws[0],
                                   jnp.where(h4 == 1, ws[1],
                                             jnp.where(h4 == 2, ws[2], ws[3])))
                plsc.store_scatter(ex_buf, [q * 16 + e * 4 + (iota & 3)],
                                   merged, mask=iota < 4)
                for k in range(8):
                    stag[row, pl.ds(k * 16, 16)] = ws[k // 2] * xls[k]
            return 0
        lax.fori_loop(0, B // 4, _quad, 0)
        pltpu.sync_copy(stag, msg_sh.at[dst_sc.at[p]], add=True)

        for k in range(B // 16):
            rows = k * 16 + iota
            colv = (dst_sc[p, pl.ds(k * 16, 16)] & 31) * 4
            for h in range(4):
                exh = plsc.load_gather(ex_buf, [rows * 4 + h])
                plsc.store_scatter(stag_den, [rows, colv + h], exh)
        pltpu.sync_copy(stag_den, den_sh.at[gd_big.at[p]], add=True)
        for k in range(B // 16):
            rows = k * 16 + iota
            colv = (dst_sc[p, pl.ds(k * 16, 16)] & 31) * 4
            for h in range(4):
                plsc.store_scatter(stag_den, [rows, colv + h], z16)
        return 0
    lax.fori_loop(0, NBLK12, _block, 0)
    plsc.subcore_barrier()

    pltpu.sync_copy(den_sh.at[pl.ds(sid * dpt, dpt)],
                    stag_den.at[pl.ds(0, dpt)])

    def _dred(q, _):
        idx = q * 16 + iota
        v = plsc.load_gather(stag_den, [idx >> 7, idx & 127])
        plsc.store_scatter(stag_den, [dpt + (idx >> 7), idx & 127],
                           1.0 / (v + 1e-16))
        return 0
    lax.fori_loop(0, 4 * RPT // 16, _dred, 0)

    def _norm(g, _):
        pltpu.sync_copy(msg_sh.at[pl.ds(r0 + g * 32, 32)],
                        stag.at[pl.ds(0, 32)])
        def _nrow(nn, _):
            for hk in range(8):
                widx = (g * 32 + nn) * 4 + hk // 2 + 0 * iota
                w = plsc.load_gather(stag_den,
                                     [dpt + (widx >> 7), widx & 127])
                m = plsc.load_gather(stag, [nn + 0 * iota, hk * 16 + iota])
                plsc.store_scatter(stag, [nn + 0 * iota, hk * 16 + iota],
                                   m * w)
            return 0
        lax.fori_loop(0, 32, _nrow, 0)
        pltpu.sync_copy(stag.at[pl.ds(0, 32)],
                        acc_hbm.at[cid, pl.ds(r0 + g * 32, 32)])
        return 0
    lax.fori_loop(0, RPT // 32, _norm, 0)


_edges8 = functools.partial(
    pl.kernel,
    out_type=jax.ShapeDtypeStruct((NC, N_PAD, 128), jnp.float32),
    scratch_types=[
        pltpu.VMEM((CHE,), jnp.int32),
        pltpu.VMEM((CHE,), jnp.int32),
        pltpu.VMEM((2, B), jnp.int32),
        pltpu.VMEM((2, B), jnp.int32),
        pltpu.VMEM((2, B), jnp.int32),
        pltpu.VMEM((2, B), jnp.int32),
        pltpu.VMEM((2 * B, 128), jnp.float32),
        pltpu.VMEM((2 * B, 128), jnp.float32),
        pltpu.VMEM((B, 128), jnp.float32),
        pltpu.VMEM((B, 128), jnp.float32),
        pltpu.VMEM((4 * B,), jnp.float32),
        pltpu.VMEM((256,), jnp.float32),
        pltpu.VMEM_SHARED((N_PAD, 128), jnp.float32),
        pltpu.VMEM_SHARED((DROWS, 128), jnp.float32),
        pltpu.SemaphoreType.DMA,
        pltpu.SemaphoreType.DMA,
        pltpu.SemaphoreType.DMA,
        pltpu.SemaphoreType.DMA,
    ],
    compiler_params=_params,
)


def _edges1_body(xl_hbm, xr_hbm, src_hbm, dst_hbm, att_hbm, acc_hbm,
                 xl3_v, xr3_v, src_v, dst_v, ga_idx, stag3, att2_v, acc_sh,
                 sem1):
    cid = lax.axis_index("c")
    sid = lax.axis_index("s")
    wid = sid * NC + cid
    iota = lax.iota(jnp.int32, 16)
    z16 = jnp.zeros((16,), jnp.float32)
    apt = AROWS3 // NS

    pltpu.sync_copy(xl_hbm, xl3_v)
    pltpu.sync_copy(xr_hbm, xr3_v)
    pltpu.sync_copy(att_hbm, att2_v)
    a20 = plsc.load_gather(att2_v, [0 * iota])
    a21 = plsc.load_gather(att2_v, [0 * iota + 1])

    def _zrow(r, _):
        for k in range(8):
            plsc.store_scatter(stag3, [r + 0 * iota, k * 16 + iota], z16)
        return 0
    lax.fori_loop(0, B3, _zrow, 0)

    pltpu.sync_copy(stag3.at[pl.ds(0, apt)],
                    acc_sh.at[pl.ds(sid * apt, apt)])
    plsc.subcore_barrier()

    def _block(g, _):
        base = wid * EPT3 + g * B3
        pltpu.sync_copy(src_hbm.at[pl.ds(base, B3)], src_v)
        pltpu.sync_copy(dst_hbm.at[pl.ds(base, B3)], dst_v)
        for k in range(B3 // 16):
            sl = pl.ds(k * 16, 16)
            ga_idx[sl] = dst_v[sl] >> 3
        for k in range(B3 // 16):
            sl = pl.ds(k * 16, 16)
            sv = src_v[sl]
            dv = dst_v[sl]
            xl0 = plsc.load_gather(xl3_v, [sv * 2])
            xl1 = plsc.load_gather(xl3_v, [sv * 2 + 1])
            xr0 = plsc.load_gather(xr3_v, [dv * 2])
            xr1 = plsc.load_gather(xr3_v, [dv * 2 + 1])
            alpha = a20 * _lrelu(xl0 + xr0) + a21 * _lrelu(xl1 + xr1)
            ids = base + k * 16 + iota
            ex = jnp.where(ids < N_EDGES, jnp.exp(alpha), 0.0)
            rows = k * 16 + iota
            colv = (dv & 7) * 16
            plsc.store_scatter(stag3, [rows, colv], ex * xl0)
            plsc.store_scatter(stag3, [rows, colv + 1], ex * xl1)
            plsc.store_scatter(stag3, [rows, colv + 2], ex)
        pltpu.sync_copy(stag3, acc_sh.at[ga_idx], add=True)
        for k in range(B3 // 16):
            sl = pl.ds(k * 16, 16)
            rows = k * 16 + iota
            colv = (dst_v[sl] & 7) * 16
            plsc.store_scatter(stag3, [rows, colv], z16)
            plsc.store_scatter(stag3, [rows, colv + 1], z16)
            plsc.store_scatter(stag3, [rows, colv + 2], z16)
        return 0
    lax.fori_loop(0, NBLK3, _block, 0)
    plsc.subcore_barrier()

    pltpu.sync_copy(acc_sh.at[pl.ds(sid * apt, apt)],
                    acc_hbm.at[cid, pl.ds(sid * apt, apt)])


_edges1 = functools.partial(
    pl.kernel,
    out_type=jax.ShapeDtypeStruct((NC, AROWS3, 128), jnp.float32),
    scratch_types=[
        pltpu.VMEM((2 * N_NODES,), jnp.float32),
        pltpu.VMEM((2 * N_NODES,), jnp.float32),
        pltpu.VMEM((B3,), jnp.int32),
        pltpu.VMEM((B3,), jnp.int32),
        pltpu.VMEM((B3,), jnp.int32),
        pltpu.VMEM((B3, 128), jnp.float32),
        pltpu.VMEM((16,), jnp.float32),
        pltpu.VMEM_SHARED((AROWS3, 128), jnp.float32),
        pltpu.SemaphoreType.DMA,
    ],
    compiler_params=_params,
)


def _mm2_body(x_ref, wl_ref, wr_ref, xl_ref, xr_ref):
    xb = x_ref[...]
    xl_ref[...] = jnp.dot(xb, wl_ref[...], preferred_element_type=jnp.float32)
    xr_ref[...] = jnp.dot(xb, wr_ref[...], preferred_element_type=jnp.float32)


def _mid_body(a0_ref, a1_ref, b_ref, wl_ref, wr_ref, xl_ref, xr_ref):
    h = jnp.concatenate([a0_ref[...], a1_ref[...]], axis=1) + b_ref[...]
    h = jnp.where(h > 0, h, jnp.exp(h) - 1.0)
    xl_ref[...] = jnp.dot(h, wl_ref[...], preferred_element_type=jnp.float32)
    xr_ref[...] = jnp.dot(h, wr_ref[...], preferred_element_type=jnp.float32)


def _fin_body(a0_ref, a1_ref, b_ref, o_ref):
    s = a0_ref[...] + a1_ref[...]
    z = s[:, 0:2] / (s[:, 2:3] + 1e-16) + b_ref[:, 0:2]
    m = jnp.max(z, axis=1, keepdims=True)
    ls = m + jnp.log(jnp.sum(jnp.exp(z - m), axis=1, keepdims=True))
    o = z - ls
    o_ref[...] = jnp.concatenate(
        [o, jnp.zeros((o.shape[0], 14), jnp.float32)], axis=1)


def _row_spec(w):
    return pl.BlockSpec((MROWS, w), lambda i: (i, 0))


def _full_spec(r, c):
    return pl.BlockSpec((r, c), lambda i: (0, 0))


def _mm2(x, wl, wr, kdim):
    f = pl.pallas_call(
        _mm2_body,
        grid=(GRID,),
        in_specs=[_row_spec(kdim), _full_spec(kdim, 256), _full_spec(kdim, 256)],
        out_specs=[_row_spec(256), _row_spec(256)],
        out_shape=[jax.ShapeDtypeStruct((N_NODES, 256), jnp.float32)] * 2,
    )
    return f(x, wl, wr)


def _mid(a0, a1, b, wl, wr):
    f = pl.pallas_call(
        _mid_body,
        grid=(GRID,),
        in_specs=[_row_spec(128), _row_spec(128), _full_spec(1, 256),
                  _full_spec(256, 256), _full_spec(256, 256)],
        out_specs=[_row_spec(256), _row_spec(256)],
        out_shape=[jax.ShapeDtypeStruct((N_NODES, 256), jnp.float32)] * 2,
    )
    return f(a0, a1, b, wl, wr)


def _fin(a0, a1, b):
    f = pl.pallas_call(
        _fin_body,
        grid=(GRID,),
        in_specs=[_row_spec(16), _row_spec(16), _full_spec(1, 16)],
        out_specs=_row_spec(16),
        out_shape=jax.ShapeDtypeStruct((N_NODES, 16), jnp.float32),
    )
    return f(a0, a1, b)


def kernel(x, edge_index, Wl1, Wr1, att1, b1, Wl3, Wr3, att3, b3,
           Wl2, Wr2, att2, b2):
    loop = jnp.arange(N_NODES, dtype=jnp.int32)
    pad = jnp.zeros((E_PAD - N_EDGES,), jnp.int32)
    src = jnp.concatenate([edge_index[0].astype(jnp.int32), loop, pad])
    dst = jnp.concatenate([edge_index[1].astype(jnp.int32), loop, pad])

    ek8 = _edges8(_edges8_body, mesh=_mesh())
    ek1 = _edges1(_edges1_body, mesh=_mesh())

    xl, xr = _mm2(x, Wl1, Wr1, 128)
    acc = ek8(xl.reshape(2 * N_NODES, 128), xr.reshape(2 * N_NODES, 128),
              src, dst, att1.reshape(-1))

    xl, xr = _mid(acc[0, :N_NODES], acc[1, :N_NODES],
                  b1.reshape(1, 256), Wl3, Wr3)
    acc = ek8(xl.reshape(2 * N_NODES, 128), xr.reshape(2 * N_NODES, 128),
              src, dst, att3.reshape(-1))

    wcat = jnp.concatenate(
        [jnp.pad(Wl2, ((0, 0), (0, 126))), jnp.pad(Wr2, ((0, 0), (0, 126)))],
        axis=1)
    xlr, _ = _mid(acc[0, :N_NODES], acc[1, :N_NODES],
                  b3.reshape(1, 256), wcat, wcat)
    xl3 = xlr[:, 0:2].reshape(-1)
    xr3 = xlr[:, 128:130].reshape(-1)
    acc3 = ek1(xl3, xr3, src, dst, jnp.pad(att2.reshape(-1), (0, 14)))
    acc3 = acc3.reshape(NC, N_PAD, 16)

    out = _fin(acc3[0, :N_NODES], acc3[1, :N_NODES],
               jnp.pad(b2, (0, 14)).reshape(1, 16))
    return out[:, :2]


# speedup vs baseline: 34.8335x; 1.0317x over previous
import functools

import jax
import jax.numpy as jnp
from jax import lax
from jax.experimental import pallas as pl
from jax.experimental.pallas import tpu as pltpu
from jax.experimental.pallas import tpu_sc as plsc

N_NODES = 10000
N_EDGES = 320000 + N_NODES
NC = 2
NS = 16
B = 48
B3 = 128
CHB = 16
CHE = CHB * B
EPT12 = 20736
E_PAD = NS * EPT12
NBLK12 = EPT12 // B
EPT3 = E_PAD // (NC * NS)
NBLK3 = EPT3 // B3
RPT = 640
N_PAD = NS * RPT
DROWS = N_PAD // 32
AROWS3 = N_PAD // 8
MROWS = 1000
GRID = N_NODES // MROWS

_params = pltpu.CompilerParams(needs_layout_passes=False)


def _mesh():
    return plsc.VectorSubcoreMesh(core_axis_name="c", subcore_axis_name="s",
                                  num_cores=NC, num_subcores=NS)


def _lrelu(v):
    return jnp.maximum(v, 0.0) + 0.2 * jnp.minimum(v, 0.0)


def _edges8_body(xl_hbm, xr_hbm, src_hbm, dst_hbm, att_hbm, acc_hbm,
                 src_ch, dst_ch, gl_big, gr_big, gd_big, dst_sc,
                 xl_rows, xr_rows, stag, stag_den, ex_buf, att_flat,
                 msg_sh, den_sh, s_xl0, s_xr0, s_xl1, s_xr1,
                 s_ms0, s_ms1, s_den):
    cid = lax.axis_index("c")
    sid = lax.axis_index("s")
    iota = lax.iota(jnp.int32, 16)
    h4 = iota & 3
    z16 = jnp.zeros((16,), jnp.float32)
    r0 = sid * RPT
    dpt = DROWS // NS

    pltpu.sync_copy(att_hbm, att_flat)
    attk = [att_flat[pl.ds(cid * 128 + k * 16, 16)] for k in range(8)]

    def _zrow(r, _):
        for k in range(8):
            plsc.store_scatter(stag, [r + 0 * iota, k * 16 + iota], z16)
            plsc.store_scatter(stag_den, [r + 0 * iota, k * 16 + iota], z16)
        return 0
    lax.fori_loop(0, B, _zrow, 0)

    def _zacc(g, _):
        pltpu.sync_copy(stag.at[pl.ds(0, 16)],
                        msg_sh.at[pl.ds(r0 + g * 16, 16)])
        return 0
    lax.fori_loop(0, RPT // 16, _zacc, 0)
    pltpu.sync_copy(stag.at[pl.ds(0, dpt)],
                    den_sh.at[pl.ds(sid * dpt, dpt)])
    plsc.subcore_barrier()

    def _refill(chunk):
        cb = sid * EPT12 + chunk * CHE
        pltpu.sync_copy(src_hbm.at[pl.ds(cb, CHE)], src_ch)
        pltpu.sync_copy(dst_hbm.at[pl.ds(cb, CHE)], dst_ch)

    def _prep(gn, pn):
        off = (gn % CHB) * B if isinstance(gn, int) else (gn & (CHB - 1)) * B
        for k in range(B // 16):
            sl = pl.ds(off + k * 16, 16)
            dl = pl.ds(k * 16, 16)
            sv = src_ch[sl]
            dv = dst_ch[sl]
            gl_big[pn, dl] = sv * 2 + cid
            gr_big[pn, dl] = dv * 2 + cid
            gd_big[pn, dl] = dv >> 5
            dst_sc[pn, dl] = dv

        @pl.when(pn == 0)
        def _():
            pltpu.async_copy(xl_hbm.at[gl_big.at[0]],
                             xl_rows.at[pl.ds(0, B)], s_xl0)
            pltpu.async_copy(xr_hbm.at[gr_big.at[0]],
                             xr_rows.at[pl.ds(0, B)], s_xr0)

        @pl.when(pn == 1)
        def _():
            pltpu.async_copy(xl_hbm.at[gl_big.at[1]],
                             xl_rows.at[pl.ds(B, B)], s_xl1)
            pltpu.async_copy(xr_hbm.at[gr_big.at[1]],
                             xr_rows.at[pl.ds(B, B)], s_xr1)

    def _wait(p):
        @pl.when(p == 0)
        def _():
            pltpu.make_async_copy(xl_hbm.at[gl_big.at[0]],
                                  xl_rows.at[pl.ds(0, B)], s_xl0).wait()
            pltpu.make_async_copy(xr_hbm.at[gr_big.at[0]],
                                  xr_rows.at[pl.ds(0, B)], s_xr0).wait()

        @pl.when(p == 1)
        def _():
            pltpu.make_async_copy(xl_hbm.at[gl_big.at[1]],
                                  xl_rows.at[pl.ds(B, B)], s_xl1).wait()
            pltpu.make_async_copy(xr_hbm.at[gr_big.at[1]],
                                  xr_rows.at[pl.ds(B, B)], s_xr1).wait()

    _refill(0)
    _prep(0, 0)

    def _msg_wait(p):
        @pl.when(p == 0)
        def _():
            pltpu.make_async_copy(stag.at[pl.ds(0, B)],
                                  msg_sh.at[dst_sc.at[0]], s_ms0).wait()

        @pl.when(p == 1)
        def _():
            pltpu.make_async_copy(stag.at[pl.ds(B, B)],
                                  msg_sh.at[dst_sc.at[1]], s_ms1).wait()

    def _block(g, _):
        p = g & 1
        gn = g + 1

        @pl.when(g >= 1)
        def _():
            pltpu.make_async_copy(stag_den, den_sh.at[gd_big.at[1 - p]],
                                  s_den).wait()
            for k in range(B // 16):
                rows = k * 16 + iota
                colv = (dst_sc[1 - p, pl.ds(k * 16, 16)] & 31) * 4
                for h in range(4):
                    plsc.store_scatter(stag_den, [rows, colv + h], z16)

        @pl.when(gn < NBLK12)
        def _():
            @pl.when((gn & (CHB - 1)) == 0)
            def _():
                _refill(gn // CHB)
            _prep(gn, gn & 1)

        @pl.when(g >= 2)
        def _():
            _msg_wait(p)
        _wait(p)
        base = sid * EPT12 + g * B

        def _quad(q, _):
            for e in range(4):
                row = q * 4 + e
                lrow = p * B + row
                ok = (base + row) < N_EDGES
                xls = []
                us = []
                for k in range(8):
                    xlk = xl_rows[lrow, pl.ds(k * 16, 16)]
                    xrk = xr_rows[lrow, pl.ds(k * 16, 16)]
                    xls.append(xlk)
                    t = _lrelu(xlk + xrk) * attk[k]
                    if k % 2 == 0:
                        us.append(t)
                    else:
                        us[k // 2] = us[k // 2] + t
                ws = []
                for h in range(4):
                    c = plsc.cumsum(us[h])
                    a_spl = jnp.take(c, jnp.full((16,), 15, jnp.int32))
                    ws.append(jnp.where(ok, jnp.exp(a_spl), 0.0))
                merged = jnp.where(h4 == 0, ws[0],
                                   jnp.where(h4 == 1, ws[1],
                                             jnp.where(h4 == 2, ws[2], ws[3])))
                plsc.store_scatter(ex_buf, [q * 16 + e * 4 + (iota & 3)],
                                   merged, mask=iota < 4)
                for k in range(8):
                    stag[lrow, pl.ds(k * 16, 16)] = ws[k // 2] * xls[k]
            return 0
        lax.fori_loop(0, B // 4, _quad, 0)

        @pl.when(p == 0)
        def _():
            pltpu.async_copy(stag.at[pl.ds(0, B)],
                             msg_sh.at[dst_sc.at[0]], s_ms0, add=True)

        @pl.when(p == 1)
        def _():
            pltpu.async_copy(stag.at[pl.ds(B, B)],
                             msg_sh.at[dst_sc.at[1]], s_ms1, add=True)

        for k in range(B // 16):
            rows = k * 16 + iota
            colv = (dst_sc[p, pl.ds(k * 16, 16)] & 31) * 4
            for h in range(4):
                exh = plsc.load_gather(ex_buf, [rows * 4 + h])
                plsc.store_scatter(stag_den, [rows, colv + h], exh)
        pltpu.async_copy(stag_den, den_sh.at[gd_big.at[p]], s_den, add=True)
        return 0
    lax.fori_loop(0, NBLK12, _block, 0)

    pltpu.make_async_copy(stag_den,
                          den_sh.at[gd_big.at[(NBLK12 - 1) & 1]], s_den).wait()
    _msg_wait(0)
    _msg_wait(1)
    plsc.subcore_barrier()

    pltpu.sync_copy(den_sh.at[pl.ds(sid * dpt, dpt)],
                    stag_den.at[pl.ds(0, dpt)])

    def _dred(q, _):
        idx = q * 16 + iota
        v = plsc.load_gather(stag_den, [idx >> 7, idx & 127])
        plsc.store_scatter(stag_den, [dpt + (idx >> 7), idx & 127],
                           1.0 / (v + 1e-16))
        return 0
    lax.fori_loop(0, 4 * RPT // 16, _dred, 0)

    def _norm(g, _):
        pltpu.sync_copy(msg_sh.at[pl.ds(r0 + g * 32, 32)],
                        stag.at[pl.ds(0, 32)])
        def _nrow(nn, _):
            for hk in range(8):
                widx = (g * 32 + nn) * 4 + hk // 2 + 0 * iota
                w = plsc.load_gather(stag_den,
                                     [dpt + (widx >> 7), widx & 127])
                m = plsc.load_gather(stag, [nn + 0 * iota, hk * 16 + iota])
                plsc.store_scatter(stag, [nn + 0 * iota, hk * 16 + iota],
                                   m * w)
            return 0
        lax.fori_loop(0, 32, _nrow, 0)
        pltpu.sync_copy(stag.at[pl.ds(0, 32)],
                        acc_hbm.at[cid, pl.ds(r0 + g * 32, 32)])
        return 0
    lax.fori_loop(0, RPT // 32, _norm, 0)


_edges8 = functools.partial(
    pl.kernel,
    out_type=jax.ShapeDtypeStruct((NC, N_PAD, 128), jnp.float32),
    scratch_types=[
        pltpu.VMEM((CHE,), jnp.int32),
        pltpu.VMEM((CHE,), jnp.int32),
        pltpu.VMEM((2, B), jnp.int32),
        pltpu.VMEM((2, B), jnp.int32),
        pltpu.VMEM((2, B), jnp.int32),
        pltpu.VMEM((2, B), jnp.int32),
        pltpu.VMEM((2 * B, 128), jnp.float32),
        pltpu.VMEM((2 * B, 128), jnp.float32),
        pltpu.VMEM((2 * B, 128), jnp.float32),
        pltpu.VMEM((B, 128), jnp.float32),
        pltpu.VMEM((4 * B,), jnp.float32),
        pltpu.VMEM((256,), jnp.float32),
        pltpu.VMEM_SHARED((N_PAD, 128), jnp.float32),
        pltpu.VMEM_SHARED((DROWS, 128), jnp.float32),
        pltpu.SemaphoreType.DMA,
        pltpu.SemaphoreType.DMA,
        pltpu.SemaphoreType.DMA,
        pltpu.SemaphoreType.DMA,
        pltpu.SemaphoreType.DMA,
        pltpu.SemaphoreType.DMA,
        pltpu.SemaphoreType.DMA,
    ],
    compiler_params=_params,
)


def _edges1_body(xl_hbm, xr_hbm, src_hbm, dst_hbm, att_hbm, acc_hbm,
                 xl3_v, xr3_v, src_v, dst_v, ga_idx, stag3, att2_v, acc_sh,
                 sem1):
    cid = lax.axis_index("c")
    sid = lax.axis_index("s")
    wid = sid * NC + cid
    iota = lax.iota(jnp.int32, 16)
    z16 = jnp.zeros((16,), jnp.float32)
    apt = AROWS3 // NS

    pltpu.sync_copy(xl_hbm, xl3_v)
    pltpu.sync_copy(xr_hbm, xr3_v)
    pltpu.sync_copy(att_hbm, att2_v)
    a20 = plsc.load_gather(att2_v, [0 * iota])
    a21 = plsc.load_gather(att2_v, [0 * iota + 1])

    def _zrow(r, _):
        for k in range(8):
            plsc.store_scatter(stag3, [r + 0 * iota, k * 16 + iota], z16)
        return 0
    lax.fori_loop(0, B3, _zrow, 0)

    pltpu.sync_copy(stag3.at[pl.ds(0, apt)],
                    acc_sh.at[pl.ds(sid * apt, apt)])
    plsc.subcore_barrier()

    def _block(g, _):
        base = wid * EPT3 + g * B3
        pltpu.sync_copy(src_hbm.at[pl.ds(base, B3)], src_v)
        pltpu.sync_copy(dst_hbm.at[pl.ds(base, B3)], dst_v)
        for k in range(B3 // 16):
            sl = pl.ds(k * 16, 16)
            ga_idx[sl] = dst_v[sl] >> 3
        for k in range(B3 // 16):
            sl = pl.ds(k * 16, 16)
            sv = src_v[sl]
            dv = dst_v[sl]
            xl0 = plsc.load_gather(xl3_v, [sv * 2])
            xl1 = plsc.load_gather(xl3_v, [sv * 2 + 1])
            xr0 = plsc.load_gather(xr3_v, [dv * 2])
            xr1 = plsc.load_gather(xr3_v, [dv * 2 + 1])
            alpha = a20 * _lrelu(xl0 + xr0) + a21 * _lrelu(xl1 + xr1)
            ids = base + k * 16 + iota
            ex = jnp.where(ids < N_EDGES, jnp.exp(alpha), 0.0)
            rows = k * 16 + iota
            colv = (dv & 7) * 16
            plsc.store_scatter(stag3, [rows, colv], ex * xl0)
            plsc.store_scatter(stag3, [rows, colv + 1], ex * xl1)
            plsc.store_scatter(stag3, [rows, colv + 2], ex)
        pltpu.sync_copy(stag3, acc_sh.at[ga_idx], add=True)
        for k in range(B3 // 16):
            sl = pl.ds(k * 16, 16)
            rows = k * 16 + iota
            colv = (dst_v[sl] & 7) * 16
            plsc.store_scatter(stag3, [rows, colv], z16)
            plsc.store_scatter(stag3, [rows, colv + 1], z16)
            plsc.store_scatter(stag3, [rows, colv + 2], z16)
        return 0
    lax.fori_loop(0, NBLK3, _block, 0)
    plsc.subcore_barrier()

    pltpu.sync_copy(acc_sh.at[pl.ds(sid * apt, apt)],
                    acc_hbm.at[cid, pl.ds(sid * apt, apt)])


_edges1 = functools.partial(
    pl.kernel,
    out_type=jax.ShapeDtypeStruct((NC, AROWS3, 128), jnp.float32),
    scratch_types=[
        pltpu.VMEM((2 * N_NODES,), jnp.float32),
        pltpu.VMEM((2 * N_NODES,), jnp.float32),
        pltpu.VMEM((B3,), jnp.int32),
        pltpu.VMEM((B3,), jnp.int32),
        pltpu.VMEM((B3,), jnp.int32),
        pltpu.VMEM((B3, 128), jnp.float32),
        pltpu.VMEM((16,), jnp.float32),
        pltpu.VMEM_SHARED((AROWS3, 128), jnp.float32),
        pltpu.SemaphoreType.DMA,
    ],
    compiler_params=_params,
)


def _mm2_body(x_ref, wl_ref, wr_ref, xl_ref, xr_ref):
    xb = x_ref[...]
    xl_ref[...] = jnp.dot(xb, wl_ref[...], preferred_element_type=jnp.float32)
    xr_ref[...] = jnp.dot(xb, wr_ref[...], preferred_element_type=jnp.float32)


def _mid_body(a0_ref, a1_ref, b_ref, wl_ref, wr_ref, xl_ref, xr_ref):
    h = jnp.concatenate([a0_ref[...], a1_ref[...]], axis=1) + b_ref[...]
    h = jnp.where(h > 0, h, jnp.exp(h) - 1.0)
    xl_ref[...] = jnp.dot(h, wl_ref[...], preferred_element_type=jnp.float32)
    xr_ref[...] = jnp.dot(h, wr_ref[...], preferred_element_type=jnp.float32)


def _fin_body(a0_ref, a1_ref, b_ref, o_ref):
    s = a0_ref[...] + a1_ref[...]
    z = s[:, 0:2] / (s[:, 2:3] + 1e-16) + b_ref[:, 0:2]
    m = jnp.max(z, axis=1, keepdims=True)
    ls = m + jnp.log(jnp.sum(jnp.exp(z - m), axis=1, keepdims=True))
    o = z - ls
    o_ref[...] = jnp.concatenate(
        [o, jnp.zeros((o.shape[0], 14), jnp.float32)], axis=1)


def _row_spec(w):
    return pl.BlockSpec((MROWS, w), lambda i: (i, 0))


def _full_spec(r, c):
    return pl.BlockSpec((r, c), lambda i: (0, 0))


def _mm2(x, wl, wr, kdim):
    f = pl.pallas_call(
        _mm2_body,
        grid=(GRID,),
        in_specs=[_row_spec(kdim), _full_spec(kdim, 256), _full_spec(kdim, 256)],
        out_specs=[_row_spec(256), _row_spec(256)],
        out_shape=[jax.ShapeDtypeStruct((N_NODES, 256), jnp.float32)] * 2,
    )
    return f(x, wl, wr)


def _mid(a0, a1, b, wl, wr):
    f = pl.pallas_call(
        _mid_body,
        grid=(GRID,),
        in_specs=[_row_spec(128), _row_spec(128), _full_spec(1, 256),
                  _full_spec(256, 256), _full_spec(256, 256)],
        out_specs=[_row_spec(256), _row_spec(256)],
        out_shape=[jax.ShapeDtypeStruct((N_NODES, 256), jnp.float32)] * 2,
    )
    return f(a0, a1, b, wl, wr)


def _fin(a0, a1, b):
    f = pl.pallas_call(
        _fin_body,
        grid=(GRID,),
        in_specs=[_row_spec(16), _row_spec(16), _full_spec(1, 16)],
        out_specs=_row_spec(16),
        out_shape=jax.ShapeDtypeStruct((N_NODES, 16), jnp.float32),
    )
    return f(a0, a1, b)


def kernel(x, edge_index, Wl1, Wr1, att1, b1, Wl3, Wr3, att3, b3,
           Wl2, Wr2, att2, b2):
    loop = jnp.arange(N_NODES, dtype=jnp.int32)
    pad = jnp.zeros((E_PAD - N_EDGES,), jnp.int32)
    src = jnp.concatenate([edge_index[0].astype(jnp.int32), loop, pad])
    dst = jnp.concatenate([edge_index[1].astype(jnp.int32), loop, pad])

    ek8 = _edges8(_edges8_body, mesh=_mesh())
    ek1 = _edges1(_edges1_body, mesh=_mesh())

    xl, xr = _mm2(x, Wl1, Wr1, 128)
    acc = ek8(xl.reshape(2 * N_NODES, 128), xr.reshape(2 * N_NODES, 128),
              src, dst, att1.reshape(-1))

    xl, xr = _mid(acc[0, :N_NODES], acc[1, :N_NODES],
                  b1.reshape(1, 256), Wl3, Wr3)
    acc = ek8(xl.reshape(2 * N_NODES, 128), xr.reshape(2 * N_NODES, 128),
              src, dst, att3.reshape(-1))

    wcat = jnp.concatenate(
        [jnp.pad(Wl2, ((0, 0), (0, 126))), jnp.pad(Wr2, ((0, 0), (0, 126)))],
        axis=1)
    xlr, _ = _mid(acc[0, :N_NODES], acc[1, :N_NODES],
                  b3.reshape(1, 256), wcat, wcat)
    xl3 = xlr[:, 0:2].reshape(-1)
    xr3 = xlr[:, 128:130].reshape(-1)
    acc3 = ek1(xl3, xr3, src, dst, jnp.pad(att2.reshape(-1), (0, 14)))
    acc3 = acc3.reshape(NC, N_PAD, 16)

    out = _fin(acc3[0, :N_NODES], acc3[1, :N_NODES],
               jnp.pad(b2, (0, 14)).reshape(1, 16))
    return out[:, :2]


# speedup vs baseline: 35.7191x; 1.0254x over previous
import functools

import jax
import jax.numpy as jnp
from jax import lax
from jax.experimental import pallas as pl
from jax.experimental.pallas import tpu as pltpu
from jax.experimental.pallas import tpu_sc as plsc

N_NODES = 10000
N_EDGES = 320000 + N_NODES
NC = 2
NS = 16
B = 48
B3 = 128
CHB = 16
CHE = CHB * B
EPT12 = 20736
E_PAD = NS * EPT12
NBLK12 = EPT12 // B
EPT3 = E_PAD // (NC * NS)
NBLK3 = EPT3 // B3
RPT = 640
N_PAD = NS * RPT
DROWS = N_PAD // 32
AROWS3 = N_PAD // 8
MROWS = 1000
GRID = N_NODES // MROWS

_params = pltpu.CompilerParams(needs_layout_passes=False)


def _mesh():
    return plsc.VectorSubcoreMesh(core_axis_name="c", subcore_axis_name="s",
                                  num_cores=NC, num_subcores=NS)


def _lrelu(v):
    return jnp.where(v > 0.0, v, 0.2 * v)


def _edges8_body(xl_hbm, xr_hbm, src_hbm, dst_hbm, att_hbm, acc_hbm,
                 src_ch, dst_ch, gl_big, gr_big, gd_big, dst_sc,
                 xl_rows, xr_rows, stag, stag_den, ex_buf, att_flat,
                 msg_sh, den_sh, s_xl0, s_xr0, s_xl1, s_xr1,
                 s_ms0, s_ms1, s_den):
    cid = lax.axis_index("c")
    sid = lax.axis_index("s")
    iota = lax.iota(jnp.int32, 16)
    h4 = iota & 3
    z16 = jnp.zeros((16,), jnp.float32)
    r0 = sid * RPT
    dpt = DROWS // NS

    pltpu.sync_copy(att_hbm, att_flat)
    attk = [att_flat[pl.ds(cid * 128 + k * 16, 16)] for k in range(8)]

    def _zrow(r, _):
        for k in range(8):
            plsc.store_scatter(stag, [r + 0 * iota, k * 16 + iota], z16)
            plsc.store_scatter(stag_den, [r + 0 * iota, k * 16 + iota], z16)
        return 0
    lax.fori_loop(0, B, _zrow, 0)

    def _zacc(g, _):
        pltpu.sync_copy(stag.at[pl.ds(0, 16)],
                        msg_sh.at[pl.ds(r0 + g * 16, 16)])
        return 0
    lax.fori_loop(0, RPT // 16, _zacc, 0)
    pltpu.sync_copy(stag.at[pl.ds(0, dpt)],
                    den_sh.at[pl.ds(sid * dpt, dpt)])
    plsc.subcore_barrier()

    def _refill(chunk):
        cb = sid * EPT12 + chunk * CHE
        pltpu.sync_copy(src_hbm.at[pl.ds(cb, CHE)], src_ch)
        pltpu.sync_copy(dst_hbm.at[pl.ds(cb, CHE)], dst_ch)

    def _prep(gn, pn):
        off = (gn % CHB) * B if isinstance(gn, int) else (gn & (CHB - 1)) * B
        for k in range(B // 16):
            sl = pl.ds(off + k * 16, 16)
            dl = pl.ds(k * 16, 16)
            sv = src_ch[sl]
            dv = dst_ch[sl]
            gl_big[pn, dl] = sv * 2 + cid
            gr_big[pn, dl] = jnp.minimum(dv, N_NODES - 1) * 2 + cid
            gd_big[pn, dl] = dv >> 5
            dst_sc[pn, dl] = dv

        @pl.when(pn == 0)
        def _():
            pltpu.async_copy(xl_hbm.at[gl_big.at[0]],
                             xl_rows.at[pl.ds(0, B)], s_xl0)
            pltpu.async_copy(xr_hbm.at[gr_big.at[0]],
                             xr_rows.at[pl.ds(0, B)], s_xr0)

        @pl.when(pn == 1)
        def _():
            pltpu.async_copy(xl_hbm.at[gl_big.at[1]],
                             xl_rows.at[pl.ds(B, B)], s_xl1)
            pltpu.async_copy(xr_hbm.at[gr_big.at[1]],
                             xr_rows.at[pl.ds(B, B)], s_xr1)

    def _wait(p):
        @pl.when(p == 0)
        def _():
            pltpu.make_async_copy(xl_hbm.at[gl_big.at[0]],
                                  xl_rows.at[pl.ds(0, B)], s_xl0).wait()
            pltpu.make_async_copy(xr_hbm.at[gr_big.at[0]],
                                  xr_rows.at[pl.ds(0, B)], s_xr0).wait()

        @pl.when(p == 1)
        def _():
            pltpu.make_async_copy(xl_hbm.at[gl_big.at[1]],
                                  xl_rows.at[pl.ds(B, B)], s_xl1).wait()
            pltpu.make_async_copy(xr_hbm.at[gr_big.at[1]],
                                  xr_rows.at[pl.ds(B, B)], s_xr1).wait()

    _refill(0)
    _prep(0, 0)

    def _msg_wait(p):
        @pl.when(p == 0)
        def _():
            pltpu.make_async_copy(stag.at[pl.ds(0, B)],
                                  msg_sh.at[dst_sc.at[0]], s_ms0).wait()

        @pl.when(p == 1)
        def _():
            pltpu.make_async_copy(stag.at[pl.ds(B, B)],
                                  msg_sh.at[dst_sc.at[1]], s_ms1).wait()

    def _block(g, _):
        p = g & 1
        gn = g + 1

        @pl.when(g >= 1)
        def _():
            pltpu.make_async_copy(stag_den, den_sh.at[gd_big.at[1 - p]],
                                  s_den).wait()
            for k in range(B // 16):
                rows = k * 16 + iota
                colv = (dst_sc[1 - p, pl.ds(k * 16, 16)] & 31) * 4
                for h in range(4):
                    plsc.store_scatter(stag_den, [rows, colv + h], z16)

        @pl.when(gn < NBLK12)
        def _():
            @pl.when((gn & (CHB - 1)) == 0)
            def _():
                _refill(gn // CHB)
            _prep(gn, gn & 1)

        @pl.when(g >= 2)
        def _():
            _msg_wait(p)
        _wait(p)
        base = sid * EPT12 + g * B

        def _quad(q, _):
            for e in range(4):
                row = q * 4 + e
                lrow = p * B + row
                xls = []
                us = []
                for k in range(8):
                    xlk = xl_rows[lrow, pl.ds(k * 16, 16)]
                    xrk = xr_rows[lrow, pl.ds(k * 16, 16)]
                    xls.append(xlk)
                    t = _lrelu(xlk + xrk) * attk[k]
                    if k % 2 == 0:
                        us.append(t)
                    else:
                        us[k // 2] = us[k // 2] + t
                ws = []
                for h in range(4):
                    c = plsc.cumsum(us[h])
                    a_spl = jnp.take(c, jnp.full((16,), 15, jnp.int32))
                    ws.append(jnp.exp(a_spl))
                merged = jnp.where(h4 == 0, ws[0],
                                   jnp.where(h4 == 1, ws[1],
                                             jnp.where(h4 == 2, ws[2], ws[3])))
                plsc.store_scatter(ex_buf, [q * 16 + e * 4 + (iota & 3)],
                                   merged, mask=iota < 4)
                for k in range(8):
                    stag[lrow, pl.ds(k * 16, 16)] = ws[k // 2] * xls[k]
            return 0
        lax.fori_loop(0, B // 4, _quad, 0)

        @pl.when(p == 0)
        def _():
            pltpu.async_copy(stag.at[pl.ds(0, B)],
                             msg_sh.at[dst_sc.at[0]], s_ms0, add=True)

        @pl.when(p == 1)
        def _():
            pltpu.async_copy(stag.at[pl.ds(B, B)],
                             msg_sh.at[dst_sc.at[1]], s_ms1, add=True)

        for k in range(B // 16):
            rows = k * 16 + iota
            colv = (dst_sc[p, pl.ds(k * 16, 16)] & 31) * 4
            for h in range(4):
                exh = plsc.load_gather(ex_buf, [rows * 4 + h])
                plsc.store_scatter(stag_den, [rows, colv + h], exh)
        pltpu.async_copy(stag_den, den_sh.at[gd_big.at[p]], s_den, add=True)
        return 0
    lax.fori_loop(0, NBLK12, _block, 0)

    pltpu.make_async_copy(stag_den,
                          den_sh.at[gd_big.at[(NBLK12 - 1) & 1]], s_den).wait()
    _msg_wait(0)
    _msg_wait(1)
    plsc.subcore_barrier()

    pltpu.sync_copy(den_sh.at[pl.ds(sid * dpt, dpt)],
                    stag_den.at[pl.ds(0, dpt)])

    def _dred(q, _):
        idx = q * 16 + iota
        v = plsc.load_gather(stag_den, [idx >> 7, idx & 127])
        plsc.store_scatter(stag_den, [dpt + (idx >> 7), idx & 127],
                           1.0 / (v + 1e-16))
        return 0
    lax.fori_loop(0, 4 * RPT // 16, _dred, 0)

    def _norm(g, _):
        pltpu.sync_copy(msg_sh.at[pl.ds(r0 + g * 32, 32)],
                        stag.at[pl.ds(0, 32)])
        def _nrow(nn, _):
            for hk in range(8):
                widx = (g * 32 + nn) * 4 + hk // 2 + 0 * iota
                w = plsc.load_gather(stag_den,
                                     [dpt + (widx >> 7), widx & 127])
                m = plsc.load_gather(stag, [nn + 0 * iota, hk * 16 + iota])
                plsc.store_scatter(stag, [nn + 0 * iota, hk * 16 + iota],
                                   m * w)
            return 0
        lax.fori_loop(0, 32, _nrow, 0)
        pltpu.sync_copy(stag.at[pl.ds(0, 32)],
                        acc_hbm.at[cid, pl.ds(r0 + g * 32, 32)])
        return 0
    lax.fori_loop(0, RPT // 32, _norm, 0)


_edges8 = functools.partial(
    pl.kernel,
    out_type=jax.ShapeDtypeStruct((NC, N_PAD, 128), jnp.float32),
    scratch_types=[
        pltpu.VMEM((CHE,), jnp.int32),
        pltpu.VMEM((CHE,), jnp.int32),
        pltpu.VMEM((2, B), jnp.int32),
        pltpu.VMEM((2, B), jnp.int32),
        pltpu.VMEM((2, B), jnp.int32),
        pltpu.VMEM((2, B), jnp.int32),
        pltpu.VMEM((2 * B, 128), jnp.float32),
        pltpu.VMEM((2 * B, 128), jnp.float32),
        pltpu.VMEM((2 * B, 128), jnp.float32),
        pltpu.VMEM((B, 128), jnp.float32),
        pltpu.VMEM((4 * B,), jnp.float32),
        pltpu.VMEM((256,), jnp.float32),
        pltpu.VMEM_SHARED((N_PAD, 128), jnp.float32),
        pltpu.VMEM_SHARED((DROWS, 128), jnp.float32),
        pltpu.SemaphoreType.DMA,
        pltpu.SemaphoreType.DMA,
        pltpu.SemaphoreType.DMA,
        pltpu.SemaphoreType.DMA,
        pltpu.SemaphoreType.DMA,
        pltpu.SemaphoreType.DMA,
        pltpu.SemaphoreType.DMA,
    ],
    compiler_params=_params,
)


def _edges1_body(xl_hbm, xr_hbm, src_hbm, dst_hbm, att_hbm, acc_hbm,
                 xl3_v, xr3_v, src_v, dst_v, ga_idx, stag3, att2_v, acc_sh,
                 sem1):
    cid = lax.axis_index("c")
    sid = lax.axis_index("s")
    wid = sid * NC + cid
    iota = lax.iota(jnp.int32, 16)
    z16 = jnp.zeros((16,), jnp.float32)
    apt = AROWS3 // NS

    pltpu.sync_copy(xl_hbm, xl3_v)
    pltpu.sync_copy(xr_hbm, xr3_v)
    pltpu.sync_copy(att_hbm, att2_v)
    a20 = plsc.load_gather(att2_v, [0 * iota])
    a21 = plsc.load_gather(att2_v, [0 * iota + 1])

    def _zrow(r, _):
        for k in range(8):
            plsc.store_scatter(stag3, [r + 0 * iota, k * 16 + iota], z16)
        return 0
    lax.fori_loop(0, B3, _zrow, 0)

    pltpu.sync_copy(stag3.at[pl.ds(0, apt)],
                    acc_sh.at[pl.ds(sid * apt, apt)])
    plsc.subcore_barrier()

    def _block(g, _):
        base = wid * EPT3 + g * B3
        pltpu.sync_copy(src_hbm.at[pl.ds(base, B3)], src_v)
        pltpu.sync_copy(dst_hbm.at[pl.ds(base, B3)], dst_v)
        for k in range(B3 // 16):
            sl = pl.ds(k * 16, 16)
            ga_idx[sl] = dst_v[sl] >> 3
        for k in range(B3 // 16):
            sl = pl.ds(k * 16, 16)
            sv = src_v[sl]
            dv = dst_v[sl]
            dvc = jnp.minimum(dv, N_NODES - 1)
            xl0 = plsc.load_gather(xl3_v, [sv * 2])
            xl1 = plsc.load_gather(xl3_v, [sv * 2 + 1])
            xr0 = plsc.load_gather(xr3_v, [dvc * 2])
            xr1 = plsc.load_gather(xr3_v, [dvc * 2 + 1])
            alpha = a20 * _lrelu(xl0 + xr0) + a21 * _lrelu(xl1 + xr1)
            ex = jnp.exp(alpha)
            rows = k * 16 + iota
            colv = (dv & 7) * 16
            plsc.store_scatter(stag3, [rows, colv], ex * xl0)
            plsc.store_scatter(stag3, [rows, colv + 1], ex * xl1)
            plsc.store_scatter(stag3, [rows, colv + 2], ex)
        pltpu.sync_copy(stag3, acc_sh.at[ga_idx], add=True)
        for k in range(B3 // 16):
            sl = pl.ds(k * 16, 16)
            rows = k * 16 + iota
            colv = (dst_v[sl] & 7) * 16
            plsc.store_scatter(stag3, [rows, colv], z16)
            plsc.store_scatter(stag3, [rows, colv + 1], z16)
            plsc.store_scatter(stag3, [rows, colv + 2], z16)
        return 0
    lax.fori_loop(0, NBLK3, _block, 0)
    plsc.subcore_barrier()

    pltpu.sync_copy(acc_sh.at[pl.ds(sid * apt, apt)],
                    acc_hbm.at[cid, pl.ds(sid * apt, apt)])


_edges1 = functools.partial(
    pl.kernel,
    out_type=jax.ShapeDtypeStruct((NC, AROWS3, 128), jnp.float32),
    scratch_types=[
        pltpu.VMEM((2 * N_NODES,), jnp.float32),
        pltpu.VMEM((2 * N_NODES,), jnp.float32),
        pltpu.VMEM((B3,), jnp.int32),
        pltpu.VMEM((B3,), jnp.int32),
        pltpu.VMEM((B3,), jnp.int32),
        pltpu.VMEM((B3, 128), jnp.float32),
        pltpu.VMEM((16,), jnp.float32),
        pltpu.VMEM_SHARED((AROWS3, 128), jnp.float32),
        pltpu.SemaphoreType.DMA,
    ],
    compiler_params=_params,
)


def _mm2_body(x_ref, wl_ref, wr_ref, xl_ref, xr_ref):
    xb = x_ref[...]
    xl_ref[...] = jnp.dot(xb, wl_ref[...], preferred_element_type=jnp.float32)
    xr_ref[...] = jnp.dot(xb, wr_ref[...], preferred_element_type=jnp.float32)


def _mid_body(a0_ref, a1_ref, b_ref, wl_ref, wr_ref, xl_ref, xr_ref):
    h = jnp.concatenate([a0_ref[...], a1_ref[...]], axis=1) + b_ref[...]
    h = jnp.where(h > 0, h, jnp.exp(h) - 1.0)
    xl_ref[...] = jnp.dot(h, wl_ref[...], preferred_element_type=jnp.float32)
    xr_ref[...] = jnp.dot(h, wr_ref[...], preferred_element_type=jnp.float32)


def _fin_body(a0_ref, a1_ref, b_ref, o_ref):
    s = a0_ref[...] + a1_ref[...]
    z = s[:, 0:2] / (s[:, 2:3] + 1e-16) + b_ref[:, 0:2]
    m = jnp.max(z, axis=1, keepdims=True)
    ls = m + jnp.log(jnp.sum(jnp.exp(z - m), axis=1, keepdims=True))
    o = z - ls
    o_ref[...] = jnp.concatenate(
        [o, jnp.zeros((o.shape[0], 14), jnp.float32)], axis=1)


def _row_spec(w):
    return pl.BlockSpec((MROWS, w), lambda i: (i, 0))


def _full_spec(r, c):
    return pl.BlockSpec((r, c), lambda i: (0, 0))


def _mm2(x, wl, wr, kdim):
    f = pl.pallas_call(
        _mm2_body,
        grid=(GRID,),
        in_specs=[_row_spec(kdim), _full_spec(kdim, 256), _full_spec(kdim, 256)],
        out_specs=[_row_spec(256), _row_spec(256)],
        out_shape=[jax.ShapeDtypeStruct((N_NODES, 256), jnp.float32)] * 2,
    )
    return f(x, wl, wr)


def _mid(a0, a1, b, wl, wr):
    f = pl.pallas_call(
        _mid_body,
        grid=(GRID,),
        in_specs=[_row_spec(128), _row_spec(128), _full_spec(1, 256),
                  _full_spec(256, 256), _full_spec(256, 256)],
        out_specs=[_row_spec(256), _row_spec(256)],
        out_shape=[jax.ShapeDtypeStruct((N_NODES, 256), jnp.float32)] * 2,
    )
    return f(a0, a1, b, wl, wr)


def _fin(a0, a1, b):
    f = pl.pallas_call(
        _fin_body,
        grid=(GRID,),
        in_specs=[_row_spec(16), _row_spec(16), _full_spec(1, 16)],
        out_specs=_row_spec(16),
        out_shape=jax.ShapeDtypeStruct((N_NODES, 16), jnp.float32),
    )
    return f(a0, a1, b)


def kernel(x, edge_index, Wl1, Wr1, att1, b1, Wl3, Wr3, att3, b3,
           Wl2, Wr2, att2, b2):
    loop = jnp.arange(N_NODES, dtype=jnp.int32)
    pad_s = jnp.zeros((E_PAD - N_EDGES,), jnp.int32)
    pad_d = jnp.full((E_PAD - N_EDGES,), 10200, jnp.int32)
    src = jnp.concatenate([edge_index[0].astype(jnp.int32), loop, pad_s])
    dst = jnp.concatenate([edge_index[1].astype(jnp.int32), loop, pad_d])

    ek8 = _edges8(_edges8_body, mesh=_mesh())
    ek1 = _edges1(_edges1_body, mesh=_mesh())

    xl, xr = _mm2(x, Wl1, Wr1, 128)
    acc = ek8(xl.reshape(2 * N_NODES, 128), xr.reshape(2 * N_NODES, 128),
              src, dst, att1.reshape(-1))

    xl, xr = _mid(acc[0, :N_NODES], acc[1, :N_NODES],
                  b1.reshape(1, 256), Wl3, Wr3)
    acc = ek8(xl.reshape(2 * N_NODES, 128), xr.reshape(2 * N_NODES, 128),
              src, dst, att3.reshape(-1))

    wcat = jnp.concatenate(
        [jnp.pad(Wl2, ((0, 0), (0, 126))), jnp.pad(Wr2, ((0, 0), (0, 126)))],
        axis=1)
    xlr, _ = _mid(acc[0, :N_NODES], acc[1, :N_NODES],
                  b3.reshape(1, 256), wcat, wcat)
    xl3 = xlr[:, 0:2].reshape(-1)
    xr3 = xlr[:, 128:130].reshape(-1)
    acc3 = ek1(xl3, xr3, src, dst, jnp.pad(att2.reshape(-1), (0, 14)))
    acc3 = acc3.reshape(NC, N_PAD, 16)

    out = _fin(acc3[0, :N_NODES], acc3[1, :N_NODES],
               jnp.pad(b2, (0, 14)).reshape(1, 16))
    return out[:, :2]


# speedup vs baseline: 35.7601x; 1.0011x over previous
import functools

import jax
import jax.numpy as jnp
from jax import lax
from jax.experimental import pallas as pl
from jax.experimental.pallas import tpu as pltpu
from jax.experimental.pallas import tpu_sc as plsc

N_NODES = 10000
N_EDGES = 320000 + N_NODES
NC = 2
NS = 16
B = 48
B3 = 128
CHB = 16
CHE = CHB * B
EPT12 = 20736
E_PAD = NS * EPT12
NBLK12 = EPT12 // B
EPT3 = E_PAD // (NC * NS)
NBLK3 = EPT3 // B3
RPT = 640
N_PAD = NS * RPT
DROWS = N_PAD // 32
AROWS3 = N_PAD // 8
MROWS = 1000
GRID = N_NODES // MROWS

_params = pltpu.CompilerParams(needs_layout_passes=False)


def _mesh():
    return plsc.VectorSubcoreMesh(core_axis_name="c", subcore_axis_name="s",
                                  num_cores=NC, num_subcores=NS)


def _lrelu(v):
    return jnp.where(v > 0.0, v, 0.2 * v)


def _edges8_body(xl_hbm, xr_hbm, src_hbm, dst_hbm, att_hbm, acc_hbm,
                 src_ch, dst_ch, gl_big, gr_big, gd_big, dst_sc,
                 xl_rows, xr_rows, stag, stag_den, ex_buf, att_flat,
                 msg_sh, den_sh, s_xl0, s_xr0, s_xl1, s_xr1,
                 s_ms0, s_ms1, s_den):
    cid = lax.axis_index("c")
    sid = lax.axis_index("s")
    iota = lax.iota(jnp.int32, 16)
    h4 = iota & 3
    z16 = jnp.zeros((16,), jnp.float32)
    r0 = sid * RPT
    dpt = DROWS // NS

    pltpu.sync_copy(att_hbm, att_flat)
    attk = [att_flat[pl.ds(cid * 128 + k * 16, 16)] for k in range(8)]

    def _zrow(r, _):
        for k in range(8):
            plsc.store_scatter(stag, [r + 0 * iota, k * 16 + iota], z16)
            plsc.store_scatter(stag_den, [r + 0 * iota, k * 16 + iota], z16)
        return 0
    lax.fori_loop(0, B, _zrow, 0)

    def _zacc(g, _):
        pltpu.sync_copy(stag.at[pl.ds(0, 16)],
                        msg_sh.at[pl.ds(r0 + g * 16, 16)])
        return 0
    lax.fori_loop(0, RPT // 16, _zacc, 0)
    pltpu.sync_copy(stag.at[pl.ds(0, dpt)],
                    den_sh.at[pl.ds(sid * dpt, dpt)])
    plsc.subcore_barrier()

    def _refill(chunk):
        cb = sid * EPT12 + chunk * CHE
        pltpu.sync_copy(src_hbm.at[pl.ds(cb, CHE)], src_ch)
        pltpu.sync_copy(dst_hbm.at[pl.ds(cb, CHE)], dst_ch)

    def _prep(gn, pn):
        off = (gn % CHB) * B if isinstance(gn, int) else (gn & (CHB - 1)) * B
        for k in range(B // 16):
            sl = pl.ds(off + k * 16, 16)
            dl = pl.ds(k * 16, 16)
            sv = src_ch[sl]
            dv = dst_ch[sl]
            gl_big[pn, dl] = sv * 2 + cid
            gr_big[pn, dl] = jnp.minimum(dv, N_NODES - 1) * 2 + cid
            gd_big[pn, dl] = dv >> 5
            dst_sc[pn, dl] = dv

        @pl.when(pn == 0)
        def _():
            pltpu.async_copy(xl_hbm.at[gl_big.at[0]],
                             xl_rows.at[pl.ds(0, B)], s_xl0)
            pltpu.async_copy(xr_hbm.at[gr_big.at[0]],
                             xr_rows.at[pl.ds(0, B)], s_xr0)

        @pl.when(pn == 1)
        def _():
            pltpu.async_copy(xl_hbm.at[gl_big.at[1]],
                             xl_rows.at[pl.ds(B, B)], s_xl1)
            pltpu.async_copy(xr_hbm.at[gr_big.at[1]],
                             xr_rows.at[pl.ds(B, B)], s_xr1)

    def _wait(p):
        @pl.when(p == 0)
        def _():
            pltpu.make_async_copy(xl_hbm.at[gl_big.at[0]],
                                  xl_rows.at[pl.ds(0, B)], s_xl0).wait()
            pltpu.make_async_copy(xr_hbm.at[gr_big.at[0]],
                                  xr_rows.at[pl.ds(0, B)], s_xr0).wait()

        @pl.when(p == 1)
        def _():
            pltpu.make_async_copy(xl_hbm.at[gl_big.at[1]],
                                  xl_rows.at[pl.ds(B, B)], s_xl1).wait()
            pltpu.make_async_copy(xr_hbm.at[gr_big.at[1]],
                                  xr_rows.at[pl.ds(B, B)], s_xr1).wait()

    _refill(0)
    _prep(0, 0)

    def _msg_wait(p):
        @pl.when(p == 0)
        def _():
            pltpu.make_async_copy(stag.at[pl.ds(0, B)],
                                  msg_sh.at[dst_sc.at[0]], s_ms0).wait()

        @pl.when(p == 1)
        def _():
            pltpu.make_async_copy(stag.at[pl.ds(B, B)],
                                  msg_sh.at[dst_sc.at[1]], s_ms1).wait()

    def _block(g, _):
        p = g & 1
        gn = g + 1

        @pl.when(g >= 1)
        def _():
            pltpu.make_async_copy(stag_den, den_sh.at[gd_big.at[1 - p]],
                                  s_den).wait()
            for k in range(B // 16):
                rows = k * 16 + iota
                colv = (dst_sc[1 - p, pl.ds(k * 16, 16)] & 31) * 4
                for h in range(4):
                    plsc.store_scatter(stag_den, [rows, colv + h], z16)

        @pl.when(gn < NBLK12)
        def _():
            @pl.when((gn & (CHB - 1)) == 0)
            def _():
                _refill(gn // CHB)
            _prep(gn, gn & 1)

        @pl.when(g >= 2)
        def _():
            _msg_wait(p)
        _wait(p)
        def _quad(q, _):
            for e in range(4):
                row = q * 4 + e
                lrow = p * B + row
                xls = []
                us = []
                for k in range(8):
                    xlk = xl_rows[lrow, pl.ds(k * 16, 16)]
                    xrk = xr_rows[lrow, pl.ds(k * 16, 16)]
                    xls.append(xlk)
                    t = _lrelu(xlk + xrk) * attk[k]
                    if k % 2 == 0:
                        us.append(t)
                    else:
                        us[k // 2] = us[k // 2] + t
                ws = []
                for h in range(4):
                    c = plsc.cumsum(us[h])
                    a_spl = jnp.take(c, jnp.full((16,), 15, jnp.int32))
                    ws.append(jnp.exp(a_spl))
                merged = jnp.where(h4 == 0, ws[0],
                                   jnp.where(h4 == 1, ws[1],
                                             jnp.where(h4 == 2, ws[2], ws[3])))
                plsc.store_scatter(ex_buf, [q * 16 + e * 4 + (iota & 3)],
                                   merged, mask=iota < 4)
                for k in range(8):
                    stag[lrow, pl.ds(k * 16, 16)] = ws[k // 2] * xls[k]
            return 0
        lax.fori_loop(0, B // 4, _quad, 0)

        @pl.when(p == 0)
        def _():
            pltpu.async_copy(stag.at[pl.ds(0, B)],
                             msg_sh.at[dst_sc.at[0]], s_ms0, add=True)

        @pl.when(p == 1)
        def _():
            pltpu.async_copy(stag.at[pl.ds(B, B)],
                             msg_sh.at[dst_sc.at[1]], s_ms1, add=True)

        for k in range(B // 16):
            rows = k * 16 + iota
            colv = (dst_sc[p, pl.ds(k * 16, 16)] & 31) * 4
            for h in range(4):
                exh = plsc.load_gather(ex_buf, [rows * 4 + h])
                plsc.store_scatter(stag_den, [rows, colv + h], exh)
        pltpu.async_copy(stag_den, den_sh.at[gd_big.at[p]], s_den, add=True)
        return 0
    lax.fori_loop(0, NBLK12, _block, 0)

    pltpu.make_async_copy(stag_den,
                          den_sh.at[gd_big.at[(NBLK12 - 1) & 1]], s_den).wait()
    _msg_wait(0)
    _msg_wait(1)
    plsc.subcore_barrier()

    pltpu.sync_copy(den_sh.at[pl.ds(sid * dpt, dpt)],
                    stag_den.at[pl.ds(0, dpt)])

    def _dred(q, _):
        idx = q * 16 + iota
        v = plsc.load_gather(stag_den, [idx >> 7, idx & 127])
        plsc.store_scatter(stag_den, [dpt + (idx >> 7), idx & 127],
                           1.0 / (v + 1e-16))
        return 0
    lax.fori_loop(0, 4 * RPT // 16, _dred, 0)

    def _norm(g, _):
        pltpu.sync_copy(msg_sh.at[pl.ds(r0 + g * 32, 32)],
                        stag.at[pl.ds(0, 32)])
        def _nrow(nn, _):
            for hk in range(8):
                widx = (g * 32 + nn) * 4 + hk // 2 + 0 * iota
                w = plsc.load_gather(stag_den,
                                     [dpt + (widx >> 7), widx & 127])
                m = plsc.load_gather(stag, [nn + 0 * iota, hk * 16 + iota])
                plsc.store_scatter(stag, [nn + 0 * iota, hk * 16 + iota],
                                   m * w)
            return 0
        lax.fori_loop(0, 32, _nrow, 0)
        pltpu.sync_copy(stag.at[pl.ds(0, 32)],
                        acc_hbm.at[cid, pl.ds(r0 + g * 32, 32)])
        return 0
    lax.fori_loop(0, RPT // 32, _norm, 0)


_edges8 = functools.partial(
    pl.kernel,
    out_type=jax.ShapeDtypeStruct((NC, N_PAD, 128), jnp.float32),
    scratch_types=[
        pltpu.VMEM((CHE,), jnp.int32),
        pltpu.VMEM((CHE,), jnp.int32),
        pltpu.VMEM((2, B), jnp.int32),
        pltpu.VMEM((2, B), jnp.int32),
        pltpu.VMEM((2, B), jnp.int32),
        pltpu.VMEM((2, B), jnp.int32),
        pltpu.VMEM((2 * B, 128), jnp.float32),
        pltpu.VMEM((2 * B, 128), jnp.float32),
        pltpu.VMEM((2 * B, 128), jnp.float32),
        pltpu.VMEM((B, 128), jnp.float32),
        pltpu.VMEM((4 * B,), jnp.float32),
        pltpu.VMEM((256,), jnp.float32),
        pltpu.VMEM_SHARED((N_PAD, 128), jnp.float32),
        pltpu.VMEM_SHARED((DROWS, 128), jnp.float32),
        pltpu.SemaphoreType.DMA,
        pltpu.SemaphoreType.DMA,
        pltpu.SemaphoreType.DMA,
        pltpu.SemaphoreType.DMA,
        pltpu.SemaphoreType.DMA,
        pltpu.SemaphoreType.DMA,
        pltpu.SemaphoreType.DMA,
    ],
    compiler_params=_params,
)


def _edges1_body(xl_hbm, xr_hbm, src_hbm, dst_hbm, att_hbm, acc_hbm,
                 xl3_v, xr3_v, src_v, dst_v, ga_idx, stag3, att2_v, acc_sh,
                 sem1):
    cid = lax.axis_index("c")
    sid = lax.axis_index("s")
    wid = sid * NC + cid
    iota = lax.iota(jnp.int32, 16)
    z16 = jnp.zeros((16,), jnp.float32)
    apt = AROWS3 // NS

    pltpu.sync_copy(xl_hbm, xl3_v)
    pltpu.sync_copy(xr_hbm, xr3_v)
    pltpu.sync_copy(att_hbm, att2_v)
    a20 = plsc.load_gather(att2_v, [0 * iota])
    a21 = plsc.load_gather(att2_v, [0 * iota + 1])

    def _zrow(r, _):
        for k in range(8):
            plsc.store_scatter(stag3, [r + 0 * iota, k * 16 + iota], z16)
        return 0
    lax.fori_loop(0, B3, _zrow, 0)

    pltpu.sync_copy(stag3.at[pl.ds(0, apt)],
                    acc_sh.at[pl.ds(sid * apt, apt)])
    plsc.subcore_barrier()

    def _block(g, _):
        base = wid * EPT3 + g * B3
        pltpu.sync_copy(src_hbm.at[pl.ds(base, B3)], src_v)
        pltpu.sync_copy(dst_hbm.at[pl.ds(base, B3)], dst_v)
        for k in range(B3 // 16):
            sl = pl.ds(k * 16, 16)
            ga_idx[sl] = dst_v[sl] >> 3
        for k in range(B3 // 16):
            sl = pl.ds(k * 16, 16)
            sv = src_v[sl]
            dv = dst_v[sl]
            dvc = jnp.minimum(dv, N_NODES - 1)
            xl0 = plsc.load_gather(xl3_v, [sv * 2])
            xl1 = plsc.load_gather(xl3_v, [sv * 2 + 1])
            xr0 = plsc.load_gather(xr3_v, [dvc * 2])
            xr1 = plsc.load_gather(xr3_v, [dvc * 2 + 1])
            alpha = a20 * _lrelu(xl0 + xr0) + a21 * _lrelu(xl1 + xr1)
            ex = jnp.exp(alpha)
            rows = k * 16 + iota
            colv = (dv & 7) * 16
            plsc.store_scatter(stag3, [rows, colv], ex * xl0)
            plsc.store_scatter(stag3, [rows, colv + 1], ex * xl1)
            plsc.store_scatter(stag3, [rows, colv + 2], ex)
        pltpu.sync_copy(stag3, acc_sh.at[ga_idx], add=True)
        for k in range(B3 // 16):
            sl = pl.ds(k * 16, 16)
            rows = k * 16 + iota
            colv = (dst_v[sl] & 7) * 16
            plsc.store_scatter(stag3, [rows, colv], z16)
            plsc.store_scatter(stag3, [rows, colv + 1], z16)
            plsc.store_scatter(stag3, [rows, colv + 2], z16)
        return 0
    lax.fori_loop(0, NBLK3, _block, 0)
    plsc.subcore_barrier()

    pltpu.sync_copy(acc_sh.at[pl.ds(sid * apt, apt)],
                    acc_hbm.at[cid, pl.ds(sid * apt, apt)])


_edges1 = functools.partial(
    pl.kernel,
    out_type=jax.ShapeDtypeStruct((NC, AROWS3, 128), jnp.float32),
    scratch_types=[
        pltpu.VMEM((2 * N_NODES,), jnp.float32),
        pltpu.VMEM((2 * N_NODES,), jnp.float32),
        pltpu.VMEM((B3,), jnp.int32),
        pltpu.VMEM((B3,), jnp.int32),
        pltpu.VMEM((B3,), jnp.int32),
        pltpu.VMEM((B3, 128), jnp.float32),
        pltpu.VMEM((16,), jnp.float32),
        pltpu.VMEM_SHARED((AROWS3, 128), jnp.float32),
        pltpu.SemaphoreType.DMA,
    ],
    compiler_params=_params,
)


def _mm2_body(x_ref, wl_ref, wr_ref, xl_ref, xr_ref):
    xb = x_ref[...]
    xl_ref[...] = jnp.dot(xb, wl_ref[...], preferred_element_type=jnp.float32)
    xr_ref[...] = jnp.dot(xb, wr_ref[...], preferred_element_type=jnp.float32)


def _mid_body(a0_ref, a1_ref, b_ref, wl_ref, wr_ref, xl_ref, xr_ref):
    h = jnp.concatenate([a0_ref[...], a1_ref[...]], axis=1) + b_ref[...]
    h = jnp.where(h > 0, h, jnp.exp(h) - 1.0)
    xl_ref[...] = jnp.dot(h, wl_ref[...], preferred_element_type=jnp.float32)
    xr_ref[...] = jnp.dot(h, wr_ref[...], preferred_element_type=jnp.float32)


def _fin_body(a0_ref, a1_ref, b_ref, o_ref):
    s = a0_ref[...] + a1_ref[...]
    z = s[:, 0:2] / (s[:, 2:3] + 1e-16) + b_ref[:, 0:2]
    m = jnp.max(z, axis=1, keepdims=True)
    ls = m + jnp.log(jnp.sum(jnp.exp(z - m), axis=1, keepdims=True))
    o = z - ls
    o_ref[...] = jnp.concatenate(
        [o, jnp.zeros((o.shape[0], 14), jnp.float32)], axis=1)


def _row_spec(w):
    return pl.BlockSpec((MROWS, w), lambda i: (i, 0))


def _full_spec(r, c):
    return pl.BlockSpec((r, c), lambda i: (0, 0))


def _mm2(x, wl, wr, kdim):
    f = pl.pallas_call(
        _mm2_body,
        grid=(GRID,),
        in_specs=[_row_spec(kdim), _full_spec(kdim, 256), _full_spec(kdim, 256)],
        out_specs=[_row_spec(256), _row_spec(256)],
        out_shape=[jax.ShapeDtypeStruct((N_NODES, 256), jnp.float32)] * 2,
    )
    return f(x, wl, wr)


def _mid(a0, a1, b, wl, wr):
    f = pl.pallas_call(
        _mid_body,
        grid=(GRID,),
        in_specs=[_row_spec(128), _row_spec(128), _full_spec(1, 256),
                  _full_spec(256, 256), _full_spec(256, 256)],
        out_specs=[_row_spec(256), _row_spec(256)],
        out_shape=[jax.ShapeDtypeStruct((N_NODES, 256), jnp.float32)] * 2,
    )
    return f(a0, a1, b, wl, wr)


def _fin(a0, a1, b):
    f = pl.pallas_call(
        _fin_body,
        grid=(GRID,),
        in_specs=[_row_spec(16), _row_spec(16), _full_spec(1, 16)],
        out_specs=_row_spec(16),
        out_shape=jax.ShapeDtypeStruct((N_NODES, 16), jnp.float32),
    )
    return f(a0, a1, b)


def kernel(x, edge_index, Wl1, Wr1, att1, b1, Wl3, Wr3, att3, b3,
           Wl2, Wr2, att2, b2):
    loop = jnp.arange(N_NODES, dtype=jnp.int32)
    pad_s = jnp.zeros((E_PAD - N_EDGES,), jnp.int32)
    pad_d = jnp.full((E_PAD - N_EDGES,), 10200, jnp.int32)
    src = jnp.concatenate([edge_index[0].astype(jnp.int32), loop, pad_s])
    dst = jnp.concatenate([edge_index[1].astype(jnp.int32), loop, pad_d])

    ek8 = _edges8(_edges8_body, mesh=_mesh())
    ek1 = _edges1(_edges1_body, mesh=_mesh())

    xl, xr = _mm2(x, Wl1, Wr1, 128)
    acc = ek8(xl.reshape(2 * N_NODES, 128), xr.reshape(2 * N_NODES, 128),
              src, dst, att1.reshape(-1))

    xl, xr = _mid(acc[0, :N_NODES], acc[1, :N_NODES],
                  b1.reshape(1, 256), Wl3, Wr3)
    acc = ek8(xl.reshape(2 * N_NODES, 128), xr.reshape(2 * N_NODES, 128),
              src, dst, att3.reshape(-1))

    wcat = jnp.concatenate(
        [jnp.pad(Wl2, ((0, 0), (0, 126))), jnp.pad(Wr2, ((0, 0), (0, 126)))],
        axis=1)
    xlr, _ = _mid(acc[0, :N_NODES], acc[1, :N_NODES],
                  b3.reshape(1, 256), wcat, wcat)
    xl3 = xlr[:, 0:2].reshape(-1)
    xr3 = xlr[:, 128:130].reshape(-1)
    acc3 = ek1(xl3, xr3, src, dst, jnp.pad(att2.reshape(-1), (0, 14)))
    acc3 = acc3.reshape(NC, N_PAD, 16)

    out = _fin(acc3[0, :N_NODES], acc3[1, :N_NODES],
               jnp.pad(b2, (0, 14)).reshape(1, 16))
    return out[:, :2]
